# Initial kernel scaffold; baseline (speedup 1.0000x reference)
import jax
import jax.numpy as jnp
from jax.experimental import pallas as pl


def kernel(x, edge_index, W1, b1, W2, b2, W3, b3, Wc, bc):
    raise NotImplementedError("write your pallas kernel here")


# speedup vs baseline: 20.8329x; 20.8329x over previous
import functools

import jax
import jax.numpy as jnp
from jax import lax
from jax.experimental import pallas as pl
from jax.experimental.pallas import tpu as pltpu
from jax.experimental.pallas import tpu_sc as plsc

NNODE = 10000
NEDGE = 320000
NP = 10240
F1 = 32
F2 = 16
FD = 16
NC, NS, L = 2, 16, 16
NW = NC * NS
CH = 128
NCHK = 80
EPAD = NW * NCHK * CH
RP = NP // NS

@functools.lru_cache(maxsize=None)
def _mesh():
  return plsc.VectorSubcoreMesh(
      core_axis_name="c", subcore_axis_name="s", num_cores=NC, num_subcores=NS)


@functools.lru_cache(maxsize=None)
def _make_deg_kernel():
  @functools.partial(
      pl.kernel,
      out_type=jax.ShapeDtypeStruct((NC, NP, FD), jnp.float32),
      mesh=_mesh(),
      compiler_params=pltpu.CompilerParams(use_tc_tiling_on_sc=False),
      scratch_types=[
          pltpu.VMEM((NCHK, CH), jnp.int32),
          pltpu.VMEM((CH, FD), jnp.float32),
          pltpu.VMEM((RP, FD), jnp.float32),
          pltpu.VMEM_SHARED((NP, FD), jnp.float32),
      ],
  )
  def deg_kernel(dst_hbm, out_hbm, dst_v, ones_v, zb, acc_s):
    c = lax.axis_index("c")
    s = lax.axis_index("s")
    wid = c * NS + s

    def fill_ones(i, carry):
      ones_v[i, pl.ds(0, L)] = jnp.full((L,), 1.0, jnp.float32)
      return carry

    lax.fori_loop(0, CH, fill_ones, 0)

    def fill_zero(i, carry):
      zb[i, pl.ds(0, L)] = jnp.zeros((L,), jnp.float32)
      return carry

    lax.fori_loop(0, RP, fill_zero, 0)
    pltpu.sync_copy(zb, acc_s.at[pl.ds(s * RP, RP)])
    plsc.subcore_barrier()

    pltpu.sync_copy(dst_hbm.at[wid], dst_v)

    def body(j, carry):
      pltpu.sync_copy(ones_v, acc_s.at[dst_v.at[j]], add=True)
      return carry

    lax.fori_loop(0, NCHK, body, 0)
    plsc.subcore_barrier()

    pltpu.sync_copy(acc_s.at[pl.ds(s * RP, RP)], zb)
    pltpu.sync_copy(zb, out_hbm.at[c].at[pl.ds(s * RP, RP)])

  return deg_kernel


@functools.lru_cache(maxsize=None)
def _make_agg_kernel(feat):
  nvec = feat // L

  @functools.partial(
      pl.kernel,
      out_type=jax.ShapeDtypeStruct((NC, NP, feat), jnp.float32),
      mesh=_mesh(),
      compiler_params=pltpu.CompilerParams(use_tc_tiling_on_sc=False),
      scratch_types=[
          pltpu.VMEM((NCHK, CH), jnp.int32),
          pltpu.VMEM((NCHK, CH), jnp.int32),
          pltpu.VMEM((CH, feat), jnp.float32),
          pltpu.VMEM((RP, feat), jnp.float32),
          pltpu.VMEM_SHARED((NP, feat), jnp.float32),
          pltpu.SemaphoreType.DMA,
      ],
  )
  def agg_kernel(h_hbm, src_hbm, dst_hbm, out_hbm, src_v, dst_v, buf, zb,
                 acc_s, sem):
    c = lax.axis_index("c")
    s = lax.axis_index("s")
    wid = c * NS + s

    def fill_zero(i, carry):
      for t in range(nvec):
        zb[i, pl.ds(t * L, L)] = jnp.zeros((L,), jnp.float32)
      return carry

    lax.fori_loop(0, RP, fill_zero, 0)
    pltpu.sync_copy(zb, acc_s.at[pl.ds(s * RP, RP)])
    plsc.subcore_barrier()

    pltpu.sync_copy(src_hbm.at[wid], src_v)
    pltpu.sync_copy(dst_hbm.at[wid], dst_v)

    def body(j, carry):
      pltpu.async_copy(h_hbm.at[src_v.at[j]], buf, sem).wait()
      pltpu.sync_copy(buf, acc_s.at[dst_v.at[j]], add=True)
      return carry

    lax.fori_loop(0, NCHK, body, 0)
    plsc.subcore_barrier()

    pltpu.sync_copy(acc_s.at[pl.ds(s * RP, RP)], zb)
    pltpu.sync_copy(zb, out_hbm.at[c].at[pl.ds(s * RP, RP)])

  return agg_kernel


BR = 1280
GRID = NP // BR


def _tc1_body(x_ref, w_ref, dp_ref, h_ref, ddi_ref):
  deg = 1.0 + dp_ref[0, :, 0:1] + dp_ref[1, :, 0:1]
  dis = lax.rsqrt(deg)
  inv = 1.0 / deg
  ht = jnp.dot(x_ref[...], w_ref[...], preferred_element_type=jnp.float32)
  h_ref[...] = ht * dis
  col = lax.broadcasted_iota(jnp.int32, (BR, FD), 1)
  ddi_ref[...] = jnp.where(col == 0, deg, jnp.where(col == 1, dis, inv))


def _tc_mid_body(ap_ref, h_ref, ddi_ref, w_ref, b_ref, out_ref):
  dis = ddi_ref[:, 1:2]
  inv = ddi_ref[:, 2:3]
  acc = ap_ref[0] + ap_ref[1] + h_ref[...]
  o = jnp.maximum(acc * dis * inv + b_ref[...], 0.0)
  out_ref[...] = jnp.dot(o, w_ref[...],
                         preferred_element_type=jnp.float32) * dis


def _tc4_body(ap_ref, h_ref, ddi_ref, b3_ref, wc_ref, bc_ref, out_ref):
  dis = ddi_ref[:, 1:2]
  acc = ap_ref[0] + ap_ref[1] + h_ref[...]
  o = jnp.maximum(acc * dis + b3_ref[...], 0.0)
  logits = jnp.dot(o, wc_ref[...],
                   preferred_element_type=jnp.float32) + bc_ref[...]
  m = jnp.max(logits, axis=1, keepdims=True)
  e = jnp.exp(logits - m)
  lse = m + jnp.log(jnp.sum(e, axis=1, keepdims=True))
  out_ref[...] = logits - lse


def _row_spec(f):
  return pl.BlockSpec((BR, f), lambda i: (i, 0))


def _part_spec(f):
  return pl.BlockSpec((NC, BR, f), lambda i: (0, i, 0))


def _full_spec(r, f):
  return pl.BlockSpec((r, f), lambda i: (0, 0))


def _pad2(a, rows, cols):
  return jnp.pad(a, ((0, rows - a.shape[0]), (0, cols - a.shape[1])))


def kernel(x, edge_index, W1, b1, W2, b2, W3, b3, Wc, bc):
  f32 = jnp.float32
  nclass = Wc.shape[0]

  xp = jnp.pad(x, ((0, NP - NNODE), (0, 0)))
  w1t = _pad2(W1.T, 128, F1)
  w2t = _pad2(W2.T, F1, F2)
  w3t = _pad2(W3.T, F2, F2)
  wct = _pad2(Wc.T, F2, nclass)
  b1p = _pad2(b1[None, :], 1, F1)
  b2p = _pad2(b2[None, :], 1, F2)
  b3p = _pad2(b3[None, :], 1, F2)
  bcp = bc[None, :]

  pad_idx = jnp.full((EPAD - NEDGE,), NNODE, jnp.int32)
  src = jnp.concatenate([edge_index[0], pad_idx]).reshape(NW, NCHK, CH)
  dst = jnp.concatenate([edge_index[1], pad_idx]).reshape(NW, NCHK, CH)

  degp = _make_deg_kernel()(dst)

  h1p, ddi = pl.pallas_call(
      _tc1_body,
      grid=(GRID,),
      in_specs=[_row_spec(128), _full_spec(128, F1), _part_spec(FD)],
      out_specs=[_row_spec(F1), _row_spec(FD)],
      out_shape=[jax.ShapeDtypeStruct((NP, F1), f32),
                 jax.ShapeDtypeStruct((NP, FD), f32)],
  )(xp, w1t, degp)

  acc1 = _make_agg_kernel(F1)(h1p, src, dst)

  h2p = pl.pallas_call(
      _tc_mid_body,
      grid=(GRID,),
      in_specs=[_part_spec(F1), _row_spec(F1), _row_spec(FD),
                _full_spec(F1, F2), _full_spec(1, F1)],
      out_specs=_row_spec(F2),
      out_shape=jax.ShapeDtypeStruct((NP, F2), f32),
  )(acc1, h1p, ddi, w2t, b1p)

  acc2 = _make_agg_kernel(F2)(h2p, src, dst)

  h3p = pl.pallas_call(
      _tc_mid_body,
      grid=(GRID,),
      in_specs=[_part_spec(F2), _row_spec(F2), _row_spec(FD),
                _full_spec(F2, F2), _full_spec(1, F2)],
      out_specs=_row_spec(F2),
      out_shape=jax.ShapeDtypeStruct((NP, F2), f32),
  )(acc2, h2p, ddi, w3t, b2p)

  acc3 = _make_agg_kernel(F2)(h3p, src, dst)

  out = pl.pallas_call(
      _tc4_body,
      grid=(GRID,),
      in_specs=[_part_spec(F2), _row_spec(F2), _row_spec(FD),
                _full_spec(1, F2), _full_spec(F2, nclass),
                _full_spec(1, nclass)],
      out_specs=_row_spec(nclass),
      out_shape=jax.ShapeDtypeStruct((NP, nclass), f32),
  )(acc3, h3p, ddi, b3p, wct, bcp)

  return out[:NNODE]


# speedup vs baseline: 27.2112x; 1.3062x over previous
import functools

import jax
import jax.numpy as jnp
from jax import lax
from jax.experimental import pallas as pl
from jax.experimental.pallas import tpu as pltpu
from jax.experimental.pallas import tpu_sc as plsc

NNODE = 10000
NEDGE = 320000
NP = 10240
F1 = 32
F2 = 16
FD = 16
NC, NS, L = 2, 16, 16
NW = NC * NS
CH = 128
NCHK = 80
EPAD = NW * NCHK * CH
RP = NP // NS

@functools.lru_cache(maxsize=None)
def _mesh():
  return plsc.VectorSubcoreMesh(
      core_axis_name="c", subcore_axis_name="s", num_cores=NC, num_subcores=NS)


NBUF = 8
NGRP = NCHK // NBUF


@functools.lru_cache(maxsize=None)
def _make_deg_kernel():
  @functools.partial(
      pl.kernel,
      out_type=jax.ShapeDtypeStruct((NC, NP, FD), jnp.float32),
      mesh=_mesh(),
      compiler_params=pltpu.CompilerParams(use_tc_tiling_on_sc=False),
      scratch_types=[
          pltpu.VMEM((NCHK, CH), jnp.int32),
          pltpu.VMEM((CH, FD), jnp.float32),
          pltpu.VMEM((RP, FD), jnp.float32),
          pltpu.VMEM_SHARED((NP, FD), jnp.float32),
      ] + [pltpu.SemaphoreType.DMA] * NBUF,
  )
  def deg_kernel(dst_hbm, out_hbm, dst_v, ones_v, zb, acc_s, *ssem):
    c = lax.axis_index("c")
    s = lax.axis_index("s")
    wid = c * NS + s

    def fill_ones(i, carry):
      ones_v[i, pl.ds(0, L)] = jnp.full((L,), 1.0, jnp.float32)
      return carry

    lax.fori_loop(0, CH, fill_ones, 0)

    def fill_zero(i, carry):
      zb[i, pl.ds(0, L)] = jnp.zeros((L,), jnp.float32)
      return carry

    lax.fori_loop(0, RP, fill_zero, 0)
    pltpu.sync_copy(zb, acc_s.at[pl.ds(s * RP, RP)])
    plsc.subcore_barrier()

    pltpu.sync_copy(dst_hbm.at[wid], dst_v)

    def scat(j, b):
      pltpu.async_copy(ones_v, acc_s.at[dst_v.at[j]], ssem[b], add=True)

    def swait(b):
      pltpu.make_async_copy(ones_v, acc_s.at[dst_v.at[0]], ssem[b]).wait()

    for i in range(NBUF):
      scat(i, i)

    def body(g, carry):
      m0 = g * NBUF
      for i in range(NBUF):
        swait(i)
        scat(m0 + i, i)
      return carry

    lax.fori_loop(1, NGRP, body, 0)
    for i in range(NBUF):
      swait(i)
    plsc.subcore_barrier()

    pltpu.sync_copy(acc_s.at[pl.ds(s * RP, RP)], zb)
    pltpu.sync_copy(zb, out_hbm.at[c].at[pl.ds(s * RP, RP)])

  return deg_kernel


@functools.lru_cache(maxsize=None)
def _make_agg_kernel(feat):
  nvec = feat // L

  @functools.partial(
      pl.kernel,
      out_type=jax.ShapeDtypeStruct((NC, NP, feat), jnp.float32),
      mesh=_mesh(),
      compiler_params=pltpu.CompilerParams(use_tc_tiling_on_sc=False),
      scratch_types=[
          pltpu.VMEM((NCHK, CH), jnp.int32),
          pltpu.VMEM((NCHK, CH), jnp.int32),
          pltpu.VMEM((NBUF, CH, feat), jnp.float32),
          pltpu.VMEM((RP, feat), jnp.float32),
          pltpu.VMEM_SHARED((NP, feat), jnp.float32),
      ] + [pltpu.SemaphoreType.DMA] * (2 * NBUF),
  )
  def agg_kernel(h_hbm, src_hbm, dst_hbm, out_hbm, src_v, dst_v, bufs, zb,
                 acc_s, *sems):
    gsem = sems[:NBUF]
    ssem = sems[NBUF:]
    c = lax.axis_index("c")
    s = lax.axis_index("s")
    wid = c * NS + s

    def fill_zero(i, carry):
      for t in range(nvec):
        zb[i, pl.ds(t * L, L)] = jnp.zeros((L,), jnp.float32)
      return carry

    lax.fori_loop(0, RP, fill_zero, 0)
    pltpu.sync_copy(zb, acc_s.at[pl.ds(s * RP, RP)])
    plsc.subcore_barrier()

    pltpu.sync_copy(src_hbm.at[wid], src_v)
    pltpu.sync_copy(dst_hbm.at[wid], dst_v)

    def gissue(j, b):
      pltpu.async_copy(h_hbm.at[src_v.at[j]], bufs.at[b], gsem[b])

    def gwait(b):
      pltpu.make_async_copy(h_hbm.at[src_v.at[0]], bufs.at[b],
                            gsem[b]).wait()

    def sissue(j, b):
      pltpu.async_copy(bufs.at[b], acc_s.at[dst_v.at[j]], ssem[b], add=True)

    def swait(b):
      pltpu.make_async_copy(bufs.at[0], acc_s.at[dst_v.at[0]],
                            ssem[b]).wait()

    PD = NBUF // 2
    for j in range(PD):
      gissue(j, j)

    def visit(m, i, first_group, last_group):
      b = i
      gwait(b)
      sissue(m, b)
      bn = (i + PD) % NBUF
      if not (first_group and i < PD):
        swait(bn)
      if not (last_group and i >= NBUF - PD):
        gissue(m + PD, bn)

    for i in range(NBUF):
      visit(i, i, True, False)

    def body(g, carry):
      m0 = g * NBUF
      for i in range(NBUF):
        visit(m0 + i, i, False, False)
      return carry

    lax.fori_loop(1, NGRP - 1, body, 0)
    for i in range(NBUF):
      visit((NGRP - 1) * NBUF + i, i, False, True)
    for b in range(PD, NBUF):
      swait(b)
    plsc.subcore_barrier()

    pltpu.sync_copy(acc_s.at[pl.ds(s * RP, RP)], zb)
    pltpu.sync_copy(zb, out_hbm.at[c].at[pl.ds(s * RP, RP)])

  return agg_kernel


BR = 1280
GRID = NP // BR


def _tc1_body(x_ref, w_ref, dp_ref, h_ref, ddi_ref):
  deg = 1.0 + dp_ref[0, :, 0:1] + dp_ref[1, :, 0:1]
  dis = lax.rsqrt(deg)
  inv = 1.0 / deg
  ht = jnp.dot(x_ref[...], w_ref[...], preferred_element_type=jnp.float32)
  h_ref[...] = ht * dis
  col = lax.broadcasted_iota(jnp.int32, (BR, FD), 1)
  ddi_ref[...] = jnp.where(col == 0, deg, jnp.where(col == 1, dis, inv))


def _tc_mid_body(ap_ref, h_ref, ddi_ref, w_ref, b_ref, out_ref):
  dis = ddi_ref[:, 1:2]
  inv = ddi_ref[:, 2:3]
  acc = ap_ref[0] + ap_ref[1] + h_ref[...]
  o = jnp.maximum(acc * dis * inv + b_ref[...], 0.0)
  out_ref[...] = jnp.dot(o, w_ref[...],
                         preferred_element_type=jnp.float32) * dis


def _tc4_body(ap_ref, h_ref, ddi_ref, b3_ref, wc_ref, bc_ref, out_ref):
  dis = ddi_ref[:, 1:2]
  acc = ap_ref[0] + ap_ref[1] + h_ref[...]
  o = jnp.maximum(acc * dis + b3_ref[...], 0.0)
  logits = jnp.dot(o, wc_ref[...],
                   preferred_element_type=jnp.float32) + bc_ref[...]
  m = jnp.max(logits, axis=1, keepdims=True)
  e = jnp.exp(logits - m)
  lse = m + jnp.log(jnp.sum(e, axis=1, keepdims=True))
  out_ref[...] = logits - lse


def _row_spec(f):
  return pl.BlockSpec((BR, f), lambda i: (i, 0))


def _part_spec(f):
  return pl.BlockSpec((NC, BR, f), lambda i: (0, i, 0))


def _full_spec(r, f):
  return pl.BlockSpec((r, f), lambda i: (0, 0))


def _pad2(a, rows, cols):
  return jnp.pad(a, ((0, rows - a.shape[0]), (0, cols - a.shape[1])))


def kernel(x, edge_index, W1, b1, W2, b2, W3, b3, Wc, bc):
  f32 = jnp.float32
  nclass = Wc.shape[0]

  xp = jnp.pad(x, ((0, NP - NNODE), (0, 0)))
  w1t = _pad2(W1.T, 128, F1)
  w2t = _pad2(W2.T, F1, F2)
  w3t = _pad2(W3.T, F2, F2)
  wct = _pad2(Wc.T, F2, nclass)
  b1p = _pad2(b1[None, :], 1, F1)
  b2p = _pad2(b2[None, :], 1, F2)
  b3p = _pad2(b3[None, :], 1, F2)
  bcp = bc[None, :]

  pad_idx = jnp.full((EPAD - NEDGE,), NNODE, jnp.int32)
  src = jnp.concatenate([edge_index[0], pad_idx]).reshape(NW, NCHK, CH)
  dst = jnp.concatenate([edge_index[1], pad_idx]).reshape(NW, NCHK, CH)

  degp = _make_deg_kernel()(dst)

  h1p, ddi = pl.pallas_call(
      _tc1_body,
      grid=(GRID,),
      in_specs=[_row_spec(128), _full_spec(128, F1), _part_spec(FD)],
      out_specs=[_row_spec(F1), _row_spec(FD)],
      out_shape=[jax.ShapeDtypeStruct((NP, F1), f32),
                 jax.ShapeDtypeStruct((NP, FD), f32)],
  )(xp, w1t, degp)

  acc1 = _make_agg_kernel(F1)(h1p, src, dst)

  h2p = pl.pallas_call(
      _tc_mid_body,
      grid=(GRID,),
      in_specs=[_part_spec(F1), _row_spec(F1), _row_spec(FD),
                _full_spec(F1, F2), _full_spec(1, F1)],
      out_specs=_row_spec(F2),
      out_shape=jax.ShapeDtypeStruct((NP, F2), f32),
  )(acc1, h1p, ddi, w2t, b1p)

  acc2 = _make_agg_kernel(F2)(h2p, src, dst)

  h3p = pl.pallas_call(
      _tc_mid_body,
      grid=(GRID,),
      in_specs=[_part_spec(F2), _row_spec(F2), _row_spec(FD),
                _full_spec(F2, F2), _full_spec(1, F2)],
      out_specs=_row_spec(F2),
      out_shape=jax.ShapeDtypeStruct((NP, F2), f32),
  )(acc2, h2p, ddi, w3t, b2p)

  acc3 = _make_agg_kernel(F2)(h3p, src, dst)

  out = pl.pallas_call(
      _tc4_body,
      grid=(GRID,),
      in_specs=[_part_spec(F2), _row_spec(F2), _row_spec(FD),
                _full_spec(1, F2), _full_spec(F2, nclass),
                _full_spec(1, nclass)],
      out_specs=_row_spec(nclass),
      out_shape=jax.ShapeDtypeStruct((NP, nclass), f32),
  )(acc3, h3p, ddi, b3p, wct, bcp)

  return out[:NNODE]


# speedup vs baseline: 31.3683x; 1.1528x over previous
import functools

import jax
import jax.numpy as jnp
from jax import lax
from jax.experimental import pallas as pl
from jax.experimental.pallas import tpu as pltpu
from jax.experimental.pallas import tpu_sc as plsc

NNODE = 10000
NEDGE = 320000
NP = 10240
F1 = 32
F2 = 8
FD = 8
NC, NS, L = 2, 16, 16
NW = NC * NS
CH = 128
NCHK = 80
EPAD = NW * NCHK * CH
RP = NP // NS

@functools.lru_cache(maxsize=None)
def _mesh():
  return plsc.VectorSubcoreMesh(
      core_axis_name="c", subcore_axis_name="s", num_cores=NC, num_subcores=NS)


NBUF = 8
NGRP = NCHK // NBUF


@functools.lru_cache(maxsize=None)
def _make_deg_kernel():
  @functools.partial(
      pl.kernel,
      out_type=jax.ShapeDtypeStruct((NC, NP, FD), jnp.float32),
      mesh=_mesh(),
      compiler_params=pltpu.CompilerParams(use_tc_tiling_on_sc=False),
      scratch_types=[
          pltpu.VMEM((NCHK, CH), jnp.int32),
          pltpu.VMEM((CH, FD), jnp.float32),
          pltpu.VMEM((RP, FD), jnp.float32),
          pltpu.VMEM_SHARED((NP, FD), jnp.float32),
      ] + [pltpu.SemaphoreType.DMA] * NBUF,
  )
  def deg_kernel(dst_hbm, zeros_hbm, ones_hbm, out_hbm, dst_v, ones_v, zb,
                 acc_s, *ssem):
    c = lax.axis_index("c")
    s = lax.axis_index("s")
    wid = c * NS + s

    pltpu.sync_copy(ones_hbm, ones_v)
    pltpu.sync_copy(zeros_hbm.at[pl.ds(s * RP, RP)], zb)
    pltpu.sync_copy(zb, acc_s.at[pl.ds(s * RP, RP)])
    plsc.subcore_barrier()

    pltpu.sync_copy(dst_hbm.at[wid], dst_v)

    def scat(j, b):
      pltpu.async_copy(ones_v, acc_s.at[dst_v.at[j]], ssem[b], add=True)

    def swait(b):
      pltpu.make_async_copy(ones_v, acc_s.at[dst_v.at[0]], ssem[b]).wait()

    for i in range(NBUF):
      scat(i, i)

    def body(g, carry):
      m0 = g * NBUF
      for i in range(NBUF):
        swait(i)
        scat(m0 + i, i)
      return carry

    lax.fori_loop(1, NGRP, body, 0)
    for i in range(NBUF):
      swait(i)
    plsc.subcore_barrier()

    pltpu.sync_copy(acc_s.at[pl.ds(s * RP, RP)], zb)
    pltpu.sync_copy(zb, out_hbm.at[c].at[pl.ds(s * RP, RP)])

  return deg_kernel


@functools.lru_cache(maxsize=None)
def _make_agg_kernel(feat):
  @functools.partial(
      pl.kernel,
      out_type=jax.ShapeDtypeStruct((NC, NP, feat), jnp.float32),
      mesh=_mesh(),
      compiler_params=pltpu.CompilerParams(use_tc_tiling_on_sc=False),
      scratch_types=[
          pltpu.VMEM((NCHK, CH), jnp.int32),
          pltpu.VMEM((NCHK, CH), jnp.int32),
          pltpu.VMEM((NBUF, CH, feat), jnp.float32),
          pltpu.VMEM((RP, feat), jnp.float32),
          pltpu.VMEM_SHARED((NP, feat), jnp.float32),
      ] + [pltpu.SemaphoreType.DMA] * (2 * NBUF),
  )
  def agg_kernel(h_hbm, src_hbm, dst_hbm, zeros_hbm, out_hbm, src_v, dst_v,
                 bufs, zb, acc_s, *sems):
    gsem = sems[:NBUF]
    ssem = sems[NBUF:]
    c = lax.axis_index("c")
    s = lax.axis_index("s")
    wid = c * NS + s

    pltpu.sync_copy(zeros_hbm.at[pl.ds(s * RP, RP)], zb)
    pltpu.sync_copy(zb, acc_s.at[pl.ds(s * RP, RP)])
    plsc.subcore_barrier()

    pltpu.sync_copy(src_hbm.at[wid], src_v)
    pltpu.sync_copy(dst_hbm.at[wid], dst_v)

    def gissue(j, b):
      pltpu.async_copy(h_hbm.at[src_v.at[j]], bufs.at[b], gsem[b])

    def gwait(b):
      pltpu.make_async_copy(h_hbm.at[src_v.at[0]], bufs.at[b],
                            gsem[b]).wait()

    def sissue(j, b):
      pltpu.async_copy(bufs.at[b], acc_s.at[dst_v.at[j]], ssem[b], add=True)

    def swait(b):
      pltpu.make_async_copy(bufs.at[0], acc_s.at[dst_v.at[0]],
                            ssem[b]).wait()

    PD = NBUF // 2
    for j in range(PD):
      gissue(j, j)

    def visit(m, i, first_group, last_group):
      b = i
      gwait(b)
      sissue(m, b)
      bn = (i + PD) % NBUF
      if not (first_group and i < PD):
        swait(bn)
      if not (last_group and i >= NBUF - PD):
        gissue(m + PD, bn)

    for i in range(NBUF):
      visit(i, i, True, False)

    def body(g, carry):
      m0 = g * NBUF
      for i in range(NBUF):
        visit(m0 + i, i, False, False)
      return carry

    lax.fori_loop(1, NGRP - 1, body, 0)
    for i in range(NBUF):
      visit((NGRP - 1) * NBUF + i, i, False, True)
    for b in range(PD, NBUF):
      swait(b)
    plsc.subcore_barrier()

    pltpu.sync_copy(acc_s.at[pl.ds(s * RP, RP)], zb)
    pltpu.sync_copy(zb, out_hbm.at[c].at[pl.ds(s * RP, RP)])

  return agg_kernel


BR = 1280
GRID = NP // BR


def _tc1_body(x_ref, w_ref, dp_ref, h_ref, ddi_ref):
  deg = 1.0 + dp_ref[0, :, 0:1] + dp_ref[1, :, 0:1]
  dis = lax.rsqrt(deg)
  inv = 1.0 / deg
  ht = jnp.dot(x_ref[...], w_ref[...], preferred_element_type=jnp.float32)
  h_ref[...] = ht * dis
  col = lax.broadcasted_iota(jnp.int32, (BR, FD), 1)
  ddi_ref[...] = jnp.where(col == 0, deg, jnp.where(col == 1, dis, inv))


def _tc_mid_body(ap_ref, h_ref, ddi_ref, w_ref, b_ref, out_ref):
  dis = ddi_ref[:, 1:2]
  inv = ddi_ref[:, 2:3]
  acc = ap_ref[0] + ap_ref[1] + h_ref[...]
  o = jnp.maximum(acc * dis * inv + b_ref[...], 0.0)
  out_ref[...] = jnp.dot(o, w_ref[...],
                         preferred_element_type=jnp.float32) * dis


def _tc4_body(ap_ref, h_ref, ddi_ref, b3_ref, wc_ref, bc_ref, out_ref):
  dis = ddi_ref[:, 1:2]
  acc = ap_ref[0] + ap_ref[1] + h_ref[...]
  o = jnp.maximum(acc * dis + b3_ref[...], 0.0)
  logits = jnp.dot(o, wc_ref[...],
                   preferred_element_type=jnp.float32) + bc_ref[...]
  m = jnp.max(logits, axis=1, keepdims=True)
  e = jnp.exp(logits - m)
  lse = m + jnp.log(jnp.sum(e, axis=1, keepdims=True))
  out_ref[...] = logits - lse


def _row_spec(f):
  return pl.BlockSpec((BR, f), lambda i: (i, 0))


def _part_spec(f):
  return pl.BlockSpec((NC, BR, f), lambda i: (0, i, 0))


def _full_spec(r, f):
  return pl.BlockSpec((r, f), lambda i: (0, 0))


def _pad2(a, rows, cols):
  return jnp.pad(a, ((0, rows - a.shape[0]), (0, cols - a.shape[1])))


def kernel(x, edge_index, W1, b1, W2, b2, W3, b3, Wc, bc):
  f32 = jnp.float32
  nclass = Wc.shape[0]

  xp = jnp.pad(x, ((0, NP - NNODE), (0, 0)))
  w1t = _pad2(W1.T, 128, F1)
  w2t = _pad2(W2.T, F1, F2)
  w3t = _pad2(W3.T, F2, F2)
  wct = _pad2(Wc.T, F2, nclass)
  b1p = _pad2(b1[None, :], 1, F1)
  b2p = _pad2(b2[None, :], 1, F2)
  b3p = _pad2(b3[None, :], 1, F2)
  bcp = bc[None, :]

  pad_idx = jnp.full((EPAD - NEDGE,), NNODE, jnp.int32)
  src = jnp.concatenate([edge_index[0], pad_idx]).reshape(NW, NCHK, CH)
  dst = jnp.concatenate([edge_index[1], pad_idx]).reshape(NW, NCHK, CH)

  deg_zeros = jnp.zeros((NP, FD), f32)
  deg_ones = jnp.ones((CH, FD), f32)
  degp = _make_deg_kernel()(dst, deg_zeros, deg_ones)

  h1p, ddi = pl.pallas_call(
      _tc1_body,
      grid=(GRID,),
      in_specs=[_row_spec(128), _full_spec(128, F1), _part_spec(FD)],
      out_specs=[_row_spec(F1), _row_spec(FD)],
      out_shape=[jax.ShapeDtypeStruct((NP, F1), f32),
                 jax.ShapeDtypeStruct((NP, FD), f32)],
  )(xp, w1t, degp)

  z1 = jnp.zeros((NP, F1), f32)
  z2 = jnp.zeros((NP, F2), f32)
  acc1 = _make_agg_kernel(F1)(h1p, src, dst, z1)

  h2p = pl.pallas_call(
      _tc_mid_body,
      grid=(GRID,),
      in_specs=[_part_spec(F1), _row_spec(F1), _row_spec(FD),
                _full_spec(F1, F2), _full_spec(1, F1)],
      out_specs=_row_spec(F2),
      out_shape=jax.ShapeDtypeStruct((NP, F2), f32),
  )(acc1, h1p, ddi, w2t, b1p)

  acc2 = _make_agg_kernel(F2)(h2p, src, dst, z2)

  h3p = pl.pallas_call(
      _tc_mid_body,
      grid=(GRID,),
      in_specs=[_part_spec(F2), _row_spec(F2), _row_spec(FD),
                _full_spec(F2, F2), _full_spec(1, F2)],
      out_specs=_row_spec(F2),
      out_shape=jax.ShapeDtypeStruct((NP, F2), f32),
  )(acc2, h2p, ddi, w3t, b2p)

  acc3 = _make_agg_kernel(F2)(h3p, src, dst, z2)

  out = pl.pallas_call(
      _tc4_body,
      grid=(GRID,),
      in_specs=[_part_spec(F2), _row_spec(F2), _row_spec(FD),
                _full_spec(1, F2), _full_spec(F2, nclass),
                _full_spec(1, nclass)],
      out_specs=_row_spec(nclass),
      out_shape=jax.ShapeDtypeStruct((NP, nclass), f32),
  )(acc3, h3p, ddi, b3p, wct, bcp)

  return out[:NNODE]


# speedup vs baseline: 49.9689x; 1.5930x over previous
import functools

import jax
import jax.numpy as jnp
from jax import lax
from jax.experimental import pallas as pl
from jax.experimental.pallas import tpu as pltpu
from jax.experimental.pallas import tpu_sc as plsc

NNODE = 10000
NEDGE = 320000
NP = 10240
F1 = 32
F2 = 8
FD = 8
NC, NS, L = 2, 16, 16
NW = NC * NS
CH = 128
NCHK = 80
EPAD = NW * NCHK * CH
RP = NP // NS

@functools.lru_cache(maxsize=None)
def _mesh():
  return plsc.VectorSubcoreMesh(
      core_axis_name="c", subcore_axis_name="s", num_cores=NC, num_subcores=NS)


NBUF = 8
NGRP = NCHK // NBUF


@functools.lru_cache(maxsize=None)
def _make_deg_kernel():
  @functools.partial(
      pl.kernel,
      out_type=jax.ShapeDtypeStruct((NC, NP, FD), jnp.float32),
      mesh=_mesh(),
      compiler_params=pltpu.CompilerParams(use_tc_tiling_on_sc=False),
      scratch_types=[
          pltpu.VMEM((NCHK, CH), jnp.int32),
          pltpu.VMEM((CH, FD), jnp.float32),
          pltpu.VMEM((RP, FD), jnp.float32),
          pltpu.VMEM_SHARED((NP, FD), jnp.float32),
      ] + [pltpu.SemaphoreType.DMA] * NBUF,
  )
  def deg_kernel(dst_hbm, zeros_hbm, ones_hbm, out_hbm, dst_v, ones_v, zb,
                 acc_s, *ssem):
    c = lax.axis_index("c")
    s = lax.axis_index("s")
    wid = c * NS + s

    pltpu.sync_copy(ones_hbm, ones_v)
    pltpu.sync_copy(zeros_hbm.at[pl.ds(s * RP, RP)], zb)
    pltpu.sync_copy(zb, acc_s.at[pl.ds(s * RP, RP)])
    plsc.subcore_barrier()

    pltpu.sync_copy(dst_hbm.at[wid], dst_v)

    def scat(j, b):
      pltpu.async_copy(ones_v, acc_s.at[dst_v.at[j]], ssem[b], add=True)

    def swait(b):
      pltpu.make_async_copy(ones_v, acc_s.at[dst_v.at[0]], ssem[b]).wait()

    for i in range(NBUF):
      scat(i, i)

    def body(g, carry):
      m0 = g * NBUF
      for i in range(NBUF):
        swait(i)
        scat(m0 + i, i)
      return carry

    lax.fori_loop(1, NGRP, body, 0)
    for i in range(NBUF):
      swait(i)
    plsc.subcore_barrier()

    pltpu.sync_copy(acc_s.at[pl.ds(s * RP, RP)], zb)
    pltpu.sync_copy(zb, out_hbm.at[c].at[pl.ds(s * RP, RP)])

  return deg_kernel


@functools.lru_cache(maxsize=None)
def _make_agg_kernel(feat):
  @functools.partial(
      pl.kernel,
      out_type=jax.ShapeDtypeStruct((NC, NP, feat), jnp.float32),
      mesh=_mesh(),
      compiler_params=pltpu.CompilerParams(use_tc_tiling_on_sc=False),
      scratch_types=[
          pltpu.VMEM((NCHK, CH), jnp.int32),
          pltpu.VMEM((NCHK, CH), jnp.int32),
          pltpu.VMEM((NBUF, CH, feat), jnp.float32),
          pltpu.VMEM((RP, feat), jnp.float32),
          pltpu.VMEM_SHARED((NP, feat), jnp.float32),
          pltpu.VMEM_SHARED((NP, feat), jnp.float32),
      ] + [pltpu.SemaphoreType.DMA] * (2 * NBUF),
  )
  def agg_kernel(h_hbm, src_hbm, dst_hbm, zeros_hbm, out_hbm, src_v, dst_v,
                 bufs, zb, acc_s, tab_s, *sems):
    gsem = sems[:NBUF]
    ssem = sems[NBUF:]
    c = lax.axis_index("c")
    s = lax.axis_index("s")
    wid = c * NS + s

    rows = pl.ds(s * RP, RP)
    pltpu.sync_copy(h_hbm.at[rows], zb)
    pltpu.sync_copy(zb, tab_s.at[rows])
    pltpu.sync_copy(zeros_hbm.at[rows], zb)
    pltpu.sync_copy(zb, acc_s.at[rows])
    plsc.subcore_barrier()

    pltpu.sync_copy(src_hbm.at[wid], src_v)
    pltpu.sync_copy(dst_hbm.at[wid], dst_v)

    def gissue(j, b):
      pltpu.async_copy(tab_s.at[src_v.at[j]], bufs.at[b], gsem[b])

    def gwait(b):
      pltpu.make_async_copy(tab_s.at[src_v.at[0]], bufs.at[b],
                            gsem[b]).wait()

    def sissue(j, b):
      pltpu.async_copy(bufs.at[b], acc_s.at[dst_v.at[j]], ssem[b], add=True)

    def swait(b):
      pltpu.make_async_copy(bufs.at[0], acc_s.at[dst_v.at[0]],
                            ssem[b]).wait()

    PD = NBUF // 2
    for j in range(PD):
      gissue(j, j)

    def visit(m, i, first_group, last_group):
      b = i
      gwait(b)
      sissue(m, b)
      bn = (i + PD) % NBUF
      if not (first_group and i < PD):
        swait(bn)
      if not (last_group and i >= NBUF - PD):
        gissue(m + PD, bn)

    for i in range(NBUF):
      visit(i, i, True, False)

    def body(g, carry):
      m0 = g * NBUF
      for i in range(NBUF):
        visit(m0 + i, i, False, False)
      return carry

    lax.fori_loop(1, NGRP - 1, body, 0)
    for i in range(NBUF):
      visit((NGRP - 1) * NBUF + i, i, False, True)
    for b in range(PD, NBUF):
      swait(b)
    plsc.subcore_barrier()

    pltpu.sync_copy(acc_s.at[pl.ds(s * RP, RP)], zb)
    pltpu.sync_copy(zb, out_hbm.at[c].at[pl.ds(s * RP, RP)])

  return agg_kernel


BR = 1280
GRID = NP // BR


def _tc1_body(x_ref, w_ref, dp_ref, h_ref, ddi_ref):
  deg = 1.0 + dp_ref[0, :, 0:1] + dp_ref[1, :, 0:1]
  dis = lax.rsqrt(deg)
  inv = 1.0 / deg
  ht = jnp.dot(x_ref[...], w_ref[...], preferred_element_type=jnp.float32)
  h_ref[...] = ht * dis
  col = lax.broadcasted_iota(jnp.int32, (BR, FD), 1)
  ddi_ref[...] = jnp.where(col == 0, deg, jnp.where(col == 1, dis, inv))


def _tc_mid_body(ap_ref, h_ref, ddi_ref, w_ref, b_ref, out_ref):
  dis = ddi_ref[:, 1:2]
  inv = ddi_ref[:, 2:3]
  acc = ap_ref[0] + ap_ref[1] + h_ref[...]
  o = jnp.maximum(acc * dis * inv + b_ref[...], 0.0)
  out_ref[...] = jnp.dot(o, w_ref[...],
                         preferred_element_type=jnp.float32) * dis


def _tc4_body(ap_ref, h_ref, ddi_ref, b3_ref, wc_ref, bc_ref, out_ref):
  dis = ddi_ref[:, 1:2]
  acc = ap_ref[0] + ap_ref[1] + h_ref[...]
  o = jnp.maximum(acc * dis + b3_ref[...], 0.0)
  logits = jnp.dot(o, wc_ref[...],
                   preferred_element_type=jnp.float32) + bc_ref[...]
  m = jnp.max(logits, axis=1, keepdims=True)
  e = jnp.exp(logits - m)
  lse = m + jnp.log(jnp.sum(e, axis=1, keepdims=True))
  out_ref[...] = logits - lse


def _row_spec(f):
  return pl.BlockSpec((BR, f), lambda i: (i, 0))


def _part_spec(f):
  return pl.BlockSpec((NC, BR, f), lambda i: (0, i, 0))


def _full_spec(r, f):
  return pl.BlockSpec((r, f), lambda i: (0, 0))


def _pad2(a, rows, cols):
  return jnp.pad(a, ((0, rows - a.shape[0]), (0, cols - a.shape[1])))


def kernel(x, edge_index, W1, b1, W2, b2, W3, b3, Wc, bc):
  f32 = jnp.float32
  nclass = Wc.shape[0]

  xp = jnp.pad(x, ((0, NP - NNODE), (0, 0)))
  w1t = _pad2(W1.T, 128, F1)
  w2t = _pad2(W2.T, F1, F2)
  w3t = _pad2(W3.T, F2, F2)
  wct = _pad2(Wc.T, F2, nclass)
  b1p = _pad2(b1[None, :], 1, F1)
  b2p = _pad2(b2[None, :], 1, F2)
  b3p = _pad2(b3[None, :], 1, F2)
  bcp = bc[None, :]

  pad_idx = jnp.full((EPAD - NEDGE,), NNODE, jnp.int32)
  src = jnp.concatenate([edge_index[0], pad_idx]).reshape(NW, NCHK, CH)
  dst = jnp.concatenate([edge_index[1], pad_idx]).reshape(NW, NCHK, CH)

  deg_zeros = jnp.zeros((NP, FD), f32)
  deg_ones = jnp.ones((CH, FD), f32)
  degp = _make_deg_kernel()(dst, deg_zeros, deg_ones)

  h1p, ddi = pl.pallas_call(
      _tc1_body,
      grid=(GRID,),
      in_specs=[_row_spec(128), _full_spec(128, F1), _part_spec(FD)],
      out_specs=[_row_spec(F1), _row_spec(FD)],
      out_shape=[jax.ShapeDtypeStruct((NP, F1), f32),
                 jax.ShapeDtypeStruct((NP, FD), f32)],
  )(xp, w1t, degp)

  z1 = jnp.zeros((NP, F1), f32)
  z2 = jnp.zeros((NP, F2), f32)
  acc1 = _make_agg_kernel(F1)(h1p, src, dst, z1)

  h2p = pl.pallas_call(
      _tc_mid_body,
      grid=(GRID,),
      in_specs=[_part_spec(F1), _row_spec(F1), _row_spec(FD),
                _full_spec(F1, F2), _full_spec(1, F1)],
      out_specs=_row_spec(F2),
      out_shape=jax.ShapeDtypeStruct((NP, F2), f32),
  )(acc1, h1p, ddi, w2t, b1p)

  acc2 = _make_agg_kernel(F2)(h2p, src, dst, z2)

  h3p = pl.pallas_call(
      _tc_mid_body,
      grid=(GRID,),
      in_specs=[_part_spec(F2), _row_spec(F2), _row_spec(FD),
                _full_spec(F2, F2), _full_spec(1, F2)],
      out_specs=_row_spec(F2),
      out_shape=jax.ShapeDtypeStruct((NP, F2), f32),
  )(acc2, h2p, ddi, w3t, b2p)

  acc3 = _make_agg_kernel(F2)(h3p, src, dst, z2)

  out = pl.pallas_call(
      _tc4_body,
      grid=(GRID,),
      in_specs=[_part_spec(F2), _row_spec(F2), _row_spec(FD),
                _full_spec(1, F2), _full_spec(F2, nclass),
                _full_spec(1, nclass)],
      out_specs=_row_spec(nclass),
      out_shape=jax.ShapeDtypeStruct((NP, nclass), f32),
  )(acc3, h3p, ddi, b3p, wct, bcp)

  return out[:NNODE]


# speedup vs baseline: 69.3546x; 1.3880x over previous
import functools

import jax
import jax.numpy as jnp
from jax import lax
from jax.experimental import pallas as pl
from jax.experimental.pallas import tpu as pltpu
from jax.experimental.pallas import tpu_sc as plsc

NNODE = 10000
NEDGE = 320000
NP = 10240
F1 = 32
F2 = 8
FD = 8
NC, NS, L = 2, 16, 16
NW = NC * NS
CH = 128
NROW = NEDGE // CH
NCHK = NROW // NW
NEXTRA = NROW - NCHK * NW
RP = NP // NS

NBUF = 6
PD = 3
NGRP = NCHK // NBUF


@functools.lru_cache(maxsize=None)
def _mesh():
  return plsc.VectorSubcoreMesh(
      core_axis_name="c", subcore_axis_name="s", num_cores=NC, num_subcores=NS)


@functools.lru_cache(maxsize=None)
def _make_deg_kernel():
  @functools.partial(
      pl.kernel,
      out_type=jax.ShapeDtypeStruct((NC, NP, 128), jnp.float32),
      mesh=_mesh(),
      compiler_params=pltpu.CompilerParams(use_tc_tiling_on_sc=False),
      scratch_types=[
          pltpu.VMEM((NCHK + 1, CH), jnp.int32),
          pltpu.VMEM((CH, FD), jnp.float32),
          pltpu.VMEM((RP, FD), jnp.float32),
          pltpu.VMEM_SHARED((NP, FD), jnp.float32),
      ] + [pltpu.SemaphoreType.DMA] * NBUF,
  )
  def deg_kernel(dst_hbm, zeros_hbm, ones_hbm, out_hbm, dst_v, ones_v, zb,
                 acc_s, *ssem):
    c = lax.axis_index("c")
    s = lax.axis_index("s")
    wid = c * NS + s
    rows = pl.ds(s * RP, RP)

    pltpu.sync_copy(ones_hbm.at[:, pl.ds(0, FD)], ones_v)
    pltpu.sync_copy(zeros_hbm.at[rows, pl.ds(0, FD)], zb)
    pltpu.sync_copy(zb, acc_s.at[rows])
    plsc.subcore_barrier()

    pltpu.sync_copy(dst_hbm.at[pl.ds(wid * NCHK, NCHK)],
                    dst_v.at[pl.ds(0, NCHK)])

    def scat(j, b):
      pltpu.async_copy(ones_v, acc_s.at[dst_v.at[j]], ssem[b], add=True)

    def swait(b):
      pltpu.make_async_copy(ones_v, acc_s.at[dst_v.at[0]], ssem[b]).wait()

    for i in range(NBUF):
      scat(i, i)

    def body(g, carry):
      m0 = g * NBUF
      for i in range(NBUF):
        swait(i)
        scat(m0 + i, i)
      return carry

    lax.fori_loop(1, NGRP, body, 0)
    for i in range(NBUF):
      swait(i)

    @pl.when(wid < NEXTRA)
    def _():
      pltpu.sync_copy(dst_hbm.at[pl.ds(NCHK * NW + wid, 1)],
                      dst_v.at[pl.ds(NCHK, 1)])
      scat(NCHK, 0)
      swait(0)

    plsc.subcore_barrier()
    pltpu.sync_copy(acc_s.at[rows], zb)
    pltpu.sync_copy(zb, out_hbm.at[c, rows, pl.ds(0, FD)])

  return deg_kernel


@functools.lru_cache(maxsize=None)
def _make_agg_kernel(feat):
  @functools.partial(
      pl.kernel,
      out_type=jax.ShapeDtypeStruct((NC, NP, 128), jnp.float32),
      mesh=_mesh(),
      compiler_params=pltpu.CompilerParams(use_tc_tiling_on_sc=False),
      scratch_types=[
          pltpu.VMEM((NCHK + 1, CH), jnp.int32),
          pltpu.VMEM((NCHK + 1, CH), jnp.int32),
          pltpu.VMEM((NBUF, CH, feat), jnp.float32),
          pltpu.VMEM((RP, feat), jnp.float32),
          pltpu.VMEM_SHARED((NP, feat), jnp.float32),
          pltpu.VMEM_SHARED((NP, feat), jnp.float32),
      ] + [pltpu.SemaphoreType.DMA] * (2 * NBUF),
  )
  def agg_kernel(h_hbm, src_hbm, dst_hbm, zeros_hbm, out_hbm, src_v, dst_v,
                 bufs, zb, acc_s, tab_s, *sems):
    gsem = sems[:NBUF]
    ssem = sems[NBUF:]
    c = lax.axis_index("c")
    s = lax.axis_index("s")
    wid = c * NS + s
    rows = pl.ds(s * RP, RP)

    pltpu.sync_copy(h_hbm.at[rows, pl.ds(0, feat)], zb)
    pltpu.sync_copy(zb, tab_s.at[rows])
    pltpu.sync_copy(zeros_hbm.at[rows, pl.ds(0, feat)], zb)
    pltpu.sync_copy(zb, acc_s.at[rows])
    plsc.subcore_barrier()

    pltpu.sync_copy(src_hbm.at[pl.ds(wid * NCHK, NCHK)],
                    src_v.at[pl.ds(0, NCHK)])
    pltpu.sync_copy(dst_hbm.at[pl.ds(wid * NCHK, NCHK)],
                    dst_v.at[pl.ds(0, NCHK)])

    def gissue(j, b):
      pltpu.async_copy(tab_s.at[src_v.at[j]], bufs.at[b], gsem[b])

    def gwait(b):
      pltpu.make_async_copy(tab_s.at[src_v.at[0]], bufs.at[b],
                            gsem[b]).wait()

    def sissue(j, b):
      pltpu.async_copy(bufs.at[b], acc_s.at[dst_v.at[j]], ssem[b], add=True)

    def swait(b):
      pltpu.make_async_copy(bufs.at[0], acc_s.at[dst_v.at[0]],
                            ssem[b]).wait()

    for j in range(PD):
      gissue(j, j)

    def visit(m, i, first_group, last_group):
      b = i
      gwait(b)
      sissue(m, b)
      bn = (i + PD) % NBUF
      if not (first_group and i < PD):
        swait(bn)
      if not (last_group and i >= NBUF - PD):
        gissue(m + PD, bn)

    for i in range(NBUF):
      visit(i, i, True, False)

    def body(g, carry):
      m0 = g * NBUF
      for i in range(NBUF):
        visit(m0 + i, i, False, False)
      return carry

    lax.fori_loop(1, NGRP - 1, body, 0)
    for i in range(NBUF):
      visit((NGRP - 1) * NBUF + i, i, False, True)
    for b in range(PD, NBUF):
      swait(b)

    @pl.when(wid < NEXTRA)
    def _():
      pltpu.sync_copy(src_hbm.at[pl.ds(NCHK * NW + wid, 1)],
                      src_v.at[pl.ds(NCHK, 1)])
      pltpu.sync_copy(dst_hbm.at[pl.ds(NCHK * NW + wid, 1)],
                      dst_v.at[pl.ds(NCHK, 1)])
      gissue(NCHK, 0)
      gwait(0)
      sissue(NCHK, 0)
      swait(0)

    plsc.subcore_barrier()
    pltpu.sync_copy(acc_s.at[rows], zb)
    pltpu.sync_copy(zb, out_hbm.at[c, rows, pl.ds(0, feat)])

  return agg_kernel


BR = 1280
GRID = NP // BR


def _tc1_body(x_ref, w_ref, dp_ref, h_ref, ddi_ref):
  deg = 1.0 + dp_ref[0, :, 0:1] + dp_ref[1, :, 0:1]
  dis = lax.rsqrt(deg)
  inv = 1.0 / deg
  ht = jnp.dot(x_ref[...], w_ref[...], preferred_element_type=jnp.float32)
  h_ref[...] = ht * dis
  col = lax.broadcasted_iota(jnp.int32, (BR, FD), 1)
  ddi_ref[...] = jnp.where(col == 0, deg, jnp.where(col == 1, dis, inv))


def _make_tc_mid_body(feat):
  def body(ap_ref, h_ref, ddi_ref, w_ref, b_ref, out_ref):
    dis = ddi_ref[:, 1:2]
    inv = ddi_ref[:, 2:3]
    col = lax.broadcasted_iota(jnp.int32, (BR, 128), 1)
    acc = jnp.where(col < feat, ap_ref[0] + ap_ref[1] + h_ref[...], 0.0)
    o = jnp.maximum(acc * dis * inv + b_ref[...], 0.0)
    out_ref[...] = jnp.dot(o, w_ref[...],
                           preferred_element_type=jnp.float32) * dis
  return body


def _tc4_body(ap_ref, h_ref, ddi_ref, b3_ref, wc_ref, bc_ref, out_ref):
  dis = ddi_ref[:, 1:2]
  col = lax.broadcasted_iota(jnp.int32, (BR, 128), 1)
  acc = jnp.where(col < F2, ap_ref[0] + ap_ref[1] + h_ref[...], 0.0)
  o = jnp.maximum(acc * dis + b3_ref[...], 0.0)
  logits = jnp.dot(o, wc_ref[...],
                   preferred_element_type=jnp.float32) + bc_ref[...]
  m = jnp.max(logits, axis=1, keepdims=True)
  e = jnp.exp(logits - m)
  lse = m + jnp.log(jnp.sum(e, axis=1, keepdims=True))
  out_ref[...] = logits - lse


def _row_spec(f):
  return pl.BlockSpec((BR, f), lambda i: (i, 0))


def _part_spec(f):
  return pl.BlockSpec((NC, BR, f), lambda i: (0, i, 0))


def _full_spec(r, f):
  return pl.BlockSpec((r, f), lambda i: (0, 0))


def _pad2(a, rows, cols):
  return jnp.pad(a, ((0, rows - a.shape[0]), (0, cols - a.shape[1])))


def kernel(x, edge_index, W1, b1, W2, b2, W3, b3, Wc, bc):
  f32 = jnp.float32
  nclass = Wc.shape[0]

  xp = jnp.pad(x, ((0, NP - NNODE), (0, 0)))
  w1t = _pad2(W1.T, 128, 128)
  w2t = _pad2(W2.T, 128, 128)
  w3t = _pad2(W3.T, 128, 128)
  wct = _pad2(Wc.T, 128, nclass)
  b1p = _pad2(b1[None, :], 1, 128)
  b2p = _pad2(b2[None, :], 1, 128)
  b3p = _pad2(b3[None, :], 1, 128)
  bcp = bc[None, :]

  src = edge_index[0].reshape(NROW, CH)
  dst = edge_index[1].reshape(NROW, CH)
  zeros128 = jnp.zeros((NP, 128), f32)
  ones128 = jnp.ones((CH, 128), f32)

  degp = _make_deg_kernel()(dst, zeros128, ones128)

  h1p, ddi = pl.pallas_call(
      _tc1_body,
      grid=(GRID,),
      in_specs=[_row_spec(128), _full_spec(128, 128), _part_spec(128)],
      out_specs=[_row_spec(128), _row_spec(FD)],
      out_shape=[jax.ShapeDtypeStruct((NP, 128), f32),
                 jax.ShapeDtypeStruct((NP, FD), f32)],
  )(xp, w1t, degp)

  acc1 = _make_agg_kernel(F1)(h1p, src, dst, zeros128)

  h2p = pl.pallas_call(
      _make_tc_mid_body(F1),
      grid=(GRID,),
      in_specs=[_part_spec(128), _row_spec(128), _row_spec(FD),
                _full_spec(128, 128), _full_spec(1, 128)],
      out_specs=_row_spec(128),
      out_shape=jax.ShapeDtypeStruct((NP, 128), f32),
  )(acc1, h1p, ddi, w2t, b1p)

  acc2 = _make_agg_kernel(F2)(h2p, src, dst, zeros128)

  h3p = pl.pallas_call(
      _make_tc_mid_body(F2),
      grid=(GRID,),
      in_specs=[_part_spec(128), _row_spec(128), _row_spec(FD),
                _full_spec(128, 128), _full_spec(1, 128)],
      out_specs=_row_spec(128),
      out_shape=jax.ShapeDtypeStruct((NP, 128), f32),
  )(acc2, h2p, ddi, w3t, b2p)

  acc3 = _make_agg_kernel(F2)(h3p, src, dst, zeros128)

  out = pl.pallas_call(
      _tc4_body,
      grid=(GRID,),
      in_specs=[_part_spec(128), _row_spec(128), _row_spec(FD),
                _full_spec(1, 128), _full_spec(128, nclass),
                _full_spec(1, nclass)],
      out_specs=_row_spec(nclass),
      out_shape=jax.ShapeDtypeStruct((NNODE, nclass), f32),
  )(acc3, h3p, ddi, b3p, wct, bcp)

  return out


# speedup vs baseline: 74.6102x; 1.0758x over previous
import functools

import jax
import jax.numpy as jnp
from jax import lax
from jax.experimental import pallas as pl
from jax.experimental.pallas import tpu as pltpu
from jax.experimental.pallas import tpu_sc as plsc

NNODE = 10000
NEDGE = 320000
NP = 10240
F1 = 32
F2 = 8
FD = 8
NC, NS, L = 2, 16, 16
NW = NC * NS
CH = 128
NROW = NEDGE // CH
NCHK = NROW // NW
NEXTRA = NROW - NCHK * NW
RP = NP // NS

NBUF = 6
PD = 3
NGRP = NCHK // NBUF


@functools.lru_cache(maxsize=None)
def _mesh():
  return plsc.VectorSubcoreMesh(
      core_axis_name="c", subcore_axis_name="s", num_cores=NC, num_subcores=NS)


@functools.lru_cache(maxsize=None)
def _make_deg_kernel():
  @functools.partial(
      pl.kernel,
      out_type=jax.ShapeDtypeStruct((NP, 128), jnp.float32),
      mesh=_mesh(),
      compiler_params=pltpu.CompilerParams(use_tc_tiling_on_sc=False),
      scratch_types=[
          pltpu.VMEM((NCHK + 1, CH), jnp.int32),
          pltpu.VMEM((CH, FD), jnp.float32),
          pltpu.VMEM((RP, FD), jnp.float32),
          pltpu.VMEM_SHARED((NP, FD), jnp.float32),
      ] + [pltpu.SemaphoreType.DMA] * NBUF,
  )
  def deg_kernel(ei_hbm, zeros_hbm, ones_hbm, out_hbm, dst_v, ones_v, zb,
                 acc_s, *ssem):
    c = lax.axis_index("c")
    s = lax.axis_index("s")
    wid = c * NS + s
    rows = pl.ds(s * RP, RP)

    pltpu.sync_copy(ones_hbm.at[:, pl.ds(0, FD)], ones_v)
    pltpu.sync_copy(zeros_hbm.at[rows, pl.ds(0, FD)], zb)
    pltpu.sync_copy(zb, acc_s.at[rows])
    plsc.subcore_barrier()

    pltpu.sync_copy(ei_hbm.at[1, pl.ds(wid * NCHK, NCHK)],
                    dst_v.at[pl.ds(0, NCHK)])

    def scat(j, b):
      pltpu.async_copy(ones_v, acc_s.at[dst_v.at[j]], ssem[b], add=True)

    def swait(b):
      pltpu.make_async_copy(ones_v, acc_s.at[dst_v.at[0]], ssem[b]).wait()

    for i in range(NBUF):
      scat(i, i)

    def body(g, carry):
      m0 = g * NBUF
      for i in range(NBUF):
        swait(i)
        scat(m0 + i, i)
      return carry

    lax.fori_loop(1, NGRP, body, 0)
    for i in range(NBUF):
      swait(i)

    @pl.when(wid < NEXTRA)
    def _():
      pltpu.sync_copy(ei_hbm.at[1, pl.ds(NCHK * NW + wid, 1)],
                      dst_v.at[pl.ds(NCHK, 1)])
      scat(NCHK, 0)
      swait(0)

    plsc.subcore_barrier()
    pltpu.sync_copy(acc_s.at[rows], zb)
    pltpu.sync_copy(zb, out_hbm.at[rows, pl.ds(c * FD, FD)])

  return deg_kernel


@functools.lru_cache(maxsize=None)
def _make_agg_kernel(feat):
  @functools.partial(
      pl.kernel,
      out_type=jax.ShapeDtypeStruct((NP, 128), jnp.float32),
      mesh=_mesh(),
      compiler_params=pltpu.CompilerParams(use_tc_tiling_on_sc=False),
      scratch_types=[
          pltpu.VMEM((NCHK + 1, CH), jnp.int32),
          pltpu.VMEM((NCHK + 1, CH), jnp.int32),
          pltpu.VMEM((NBUF, CH, feat), jnp.float32),
          pltpu.VMEM((RP, feat), jnp.float32),
          pltpu.VMEM_SHARED((NP, feat), jnp.float32),
          pltpu.VMEM_SHARED((NP, feat), jnp.float32),
      ] + [pltpu.SemaphoreType.DMA] * (2 * NBUF),
  )
  def agg_kernel(h_hbm, ei_hbm, zeros_hbm, out_hbm, src_v, dst_v,
                 bufs, zb, acc_s, tab_s, *sems):
    gsem = sems[:NBUF]
    ssem = sems[NBUF:]
    c = lax.axis_index("c")
    s = lax.axis_index("s")
    wid = c * NS + s
    rows = pl.ds(s * RP, RP)

    pltpu.sync_copy(h_hbm.at[rows, pl.ds(0, feat)], zb)
    pltpu.sync_copy(zb, tab_s.at[rows])
    pltpu.sync_copy(zeros_hbm.at[rows, pl.ds(0, feat)], zb)
    pltpu.sync_copy(zb, acc_s.at[rows])
    plsc.subcore_barrier()

    pltpu.sync_copy(ei_hbm.at[0, pl.ds(wid * NCHK, NCHK)],
                    src_v.at[pl.ds(0, NCHK)])
    pltpu.sync_copy(ei_hbm.at[1, pl.ds(wid * NCHK, NCHK)],
                    dst_v.at[pl.ds(0, NCHK)])

    def gissue(j, b):
      pltpu.async_copy(tab_s.at[src_v.at[j]], bufs.at[b], gsem[b])

    def gwait(b):
      pltpu.make_async_copy(tab_s.at[src_v.at[0]], bufs.at[b],
                            gsem[b]).wait()

    def sissue(j, b):
      pltpu.async_copy(bufs.at[b], acc_s.at[dst_v.at[j]], ssem[b], add=True)

    def swait(b):
      pltpu.make_async_copy(bufs.at[0], acc_s.at[dst_v.at[0]],
                            ssem[b]).wait()

    for j in range(PD):
      gissue(j, j)

    def visit(m, i, first_group, last_group):
      b = i
      gwait(b)
      sissue(m, b)
      bn = (i + PD) % NBUF
      if not (first_group and i < PD):
        swait(bn)
      if not (last_group and i >= NBUF - PD):
        gissue(m + PD, bn)

    for i in range(NBUF):
      visit(i, i, True, False)

    def body(g, carry):
      m0 = g * NBUF
      for i in range(NBUF):
        visit(m0 + i, i, False, False)
      return carry

    lax.fori_loop(1, NGRP - 1, body, 0)
    for i in range(NBUF):
      visit((NGRP - 1) * NBUF + i, i, False, True)
    for b in range(PD, NBUF):
      swait(b)

    @pl.when(wid < NEXTRA)
    def _():
      pltpu.sync_copy(ei_hbm.at[0, pl.ds(NCHK * NW + wid, 1)],
                      src_v.at[pl.ds(NCHK, 1)])
      pltpu.sync_copy(ei_hbm.at[1, pl.ds(NCHK * NW + wid, 1)],
                      dst_v.at[pl.ds(NCHK, 1)])
      gissue(NCHK, 0)
      gwait(0)
      sissue(NCHK, 0)
      swait(0)

    plsc.subcore_barrier()
    pltpu.sync_copy(acc_s.at[rows], zb)
    pltpu.sync_copy(zb, out_hbm.at[rows, pl.ds(c * feat, feat)])

  return agg_kernel


BR = 1280
GRID = NP // BR
def _tc1_body(x_ref, w_ref, dp_ref, h_ref, ddi_ref):
  deg = 1.0 + dp_ref[:, 0:1] + dp_ref[:, FD:FD + 1]
  dis = lax.rsqrt(deg)
  inv = 1.0 / deg
  ht = jnp.dot(x_ref[...], w_ref[...], preferred_element_type=jnp.float32)
  h_ref[...] = ht * dis
  col = lax.broadcasted_iota(jnp.int32, (BR, FD), 1)
  ddi_ref[...] = jnp.where(col == 0, deg, jnp.where(col == 1, dis, inv))


def _make_tc_mid_body(fin):
  def body(ap_ref, h_ref, ddi_ref, w_ref, b_ref, out_ref):
    dis = ddi_ref[:, 1:2]
    inv = ddi_ref[:, 2:3]
    acc = ap_ref[:, 0:fin] + ap_ref[:, fin:2 * fin] + h_ref[:, 0:fin]
    o = jnp.maximum(acc * dis * inv + b_ref[...], 0.0)
    out_ref[...] = jnp.dot(o, w_ref[...],
                           preferred_element_type=jnp.float32) * dis
  return body


def _tc4_body(ap_ref, h_ref, ddi_ref, b3_ref, wc_ref, bc_ref, out_ref):
  dis = ddi_ref[:, 1:2]
  acc = ap_ref[:, 0:F2] + ap_ref[:, F2:2 * F2] + h_ref[:, 0:F2]
  o = jnp.maximum(acc * dis + b3_ref[...], 0.0)
  logits = jnp.dot(o, wc_ref[...],
                   preferred_element_type=jnp.float32) + bc_ref[...]
  m = jnp.max(logits, axis=1, keepdims=True)
  e = jnp.exp(logits - m)
  lse = m + jnp.log(jnp.sum(e, axis=1, keepdims=True))
  out_ref[...] = logits - lse


def _row_spec(f):
  return pl.BlockSpec((BR, f), lambda i: (i, 0))


def _full_spec(r, f):
  return pl.BlockSpec((r, f), lambda i: (0, 0))


def _pad2(a, rows, cols):
  return jnp.pad(a, ((0, rows - a.shape[0]), (0, cols - a.shape[1])))


def kernel(x, edge_index, W1, b1, W2, b2, W3, b3, Wc, bc):
  f32 = jnp.float32
  nclass = Wc.shape[0]

  xp = jnp.pad(x, ((0, NP - NNODE), (0, 0)))
  w1t = _pad2(W1.T, 128, 128)
  w2t = _pad2(W2.T, F1, 128)
  w3t = _pad2(W3.T, F2, 128)
  wct = _pad2(Wc.T, F2, nclass)
  b1p = _pad2(b1[None, :], 1, F1)
  b2p = _pad2(b2[None, :], 1, F2)
  b3p = _pad2(b3[None, :], 1, F2)
  bcp = bc[None, :]

  ei = edge_index.reshape(2, NROW, CH)
  zeros128 = jnp.zeros((NP, 128), f32)
  ones128 = jnp.ones((CH, 128), f32)

  degp = _make_deg_kernel()(ei, zeros128, ones128)

  h1p, ddi = pl.pallas_call(
      _tc1_body,
      grid=(GRID,),
      in_specs=[_row_spec(128), _full_spec(128, 128), _row_spec(128)],
      out_specs=[_row_spec(128), _row_spec(FD)],
      out_shape=[jax.ShapeDtypeStruct((NP, 128), f32),
                 jax.ShapeDtypeStruct((NP, FD), f32)],
  )(xp, w1t, degp)

  acc1 = _make_agg_kernel(F1)(h1p, ei, zeros128)

  h2p = pl.pallas_call(
      _make_tc_mid_body(F1),
      grid=(GRID,),
      in_specs=[_row_spec(128), _row_spec(128), _row_spec(FD),
                _full_spec(F1, 128), _full_spec(1, F1)],
      out_specs=_row_spec(128),
      out_shape=jax.ShapeDtypeStruct((NP, 128), f32),
  )(acc1, h1p, ddi, w2t, b1p)

  acc2 = _make_agg_kernel(F2)(h2p, ei, zeros128)

  h3p = pl.pallas_call(
      _make_tc_mid_body(F2),
      grid=(GRID,),
      in_specs=[_row_spec(128), _row_spec(128), _row_spec(FD),
                _full_spec(F2, 128), _full_spec(1, F2)],
      out_specs=_row_spec(128),
      out_shape=jax.ShapeDtypeStruct((NP, 128), f32),
  )(acc2, h2p, ddi, w3t, b2p)

  acc3 = _make_agg_kernel(F2)(h3p, ei, zeros128)

  out = pl.pallas_call(
      _tc4_body,
      grid=(GRID,),
      in_specs=[_row_spec(128), _row_spec(128), _row_spec(FD),
                _full_spec(1, F2), _full_spec(F2, nclass),
                _full_spec(1, nclass)],
      out_specs=_row_spec(nclass),
      out_shape=jax.ShapeDtypeStruct((NNODE, nclass), f32),
  )(acc3, h3p, ddi, b3p, wct, bcp)

  return out


# speedup vs baseline: 79.9443x; 1.0715x over previous
import functools

import jax
import jax.numpy as jnp
from jax import lax
from jax.experimental import pallas as pl
from jax.experimental.pallas import tpu as pltpu
from jax.experimental.pallas import tpu_sc as plsc

NNODE = 10000
NEDGE = 320000
NP = 10240
F1 = 32
F2 = 8
FD = 8
NC, NS, L = 2, 16, 16
NW = NC * NS
CH = 128
NROW = NEDGE // CH
NCHK = NROW // NW
NEXTRA = NROW - NCHK * NW
RP = NP // NS

NBUF = 6
PD = 3
NGRP = NCHK // NBUF


@functools.lru_cache(maxsize=None)
def _mesh():
  return plsc.VectorSubcoreMesh(
      core_axis_name="c", subcore_axis_name="s", num_cores=NC, num_subcores=NS)


@functools.lru_cache(maxsize=None)
def _make_deg_kernel():
  @functools.partial(
      pl.kernel,
      out_type=jax.ShapeDtypeStruct((NP, 128), jnp.float32),
      mesh=_mesh(),
      compiler_params=pltpu.CompilerParams(use_tc_tiling_on_sc=False),
      scratch_types=[
          pltpu.VMEM((NCHK + 1, CH), jnp.int32),
          pltpu.VMEM((CH, FD), jnp.float32),
          pltpu.VMEM((RP, FD), jnp.float32),
          pltpu.VMEM_SHARED((NP, FD), jnp.float32),
      ] + [pltpu.SemaphoreType.DMA] * NBUF,
  )
  def deg_kernel(ei_hbm, zeros_hbm, ones_hbm, out_hbm, dst_v, ones_v, zb,
                 acc_s, *ssem):
    c = lax.axis_index("c")
    s = lax.axis_index("s")
    wid = c * NS + s
    rows = pl.ds(s * RP, RP)

    pltpu.sync_copy(ones_hbm.at[:, pl.ds(0, FD)], ones_v)
    pltpu.sync_copy(zeros_hbm.at[:, pl.ds(0, FD)], zb)
    pltpu.sync_copy(zb, acc_s.at[rows])
    plsc.subcore_barrier()

    pltpu.sync_copy(ei_hbm.at[1, pl.ds(wid * NCHK, NCHK)],
                    dst_v.at[pl.ds(0, NCHK)])

    def scat(j, b):
      pltpu.async_copy(ones_v, acc_s.at[dst_v.at[j]], ssem[b], add=True)

    def swait(b):
      pltpu.make_async_copy(ones_v, acc_s.at[dst_v.at[0]], ssem[b]).wait()

    for i in range(NBUF):
      scat(i, i)

    def body(g, carry):
      m0 = g * NBUF
      for i in range(NBUF):
        swait(i)
        scat(m0 + i, i)
      return carry

    lax.fori_loop(1, NGRP, body, 0)
    for i in range(NBUF):
      swait(i)

    @pl.when(wid < NEXTRA)
    def _():
      pltpu.sync_copy(ei_hbm.at[1, pl.ds(NCHK * NW + wid, 1)],
                      dst_v.at[pl.ds(NCHK, 1)])
      scat(NCHK, 0)
      swait(0)

    plsc.subcore_barrier()
    pltpu.sync_copy(acc_s.at[rows], zb)
    pltpu.sync_copy(zb, out_hbm.at[rows, pl.ds(c * FD, FD)])

  return deg_kernel


@functools.lru_cache(maxsize=None)
def _make_agg_kernel(feat):
  @functools.partial(
      pl.kernel,
      out_type=jax.ShapeDtypeStruct((NP, 128), jnp.float32),
      mesh=_mesh(),
      compiler_params=pltpu.CompilerParams(use_tc_tiling_on_sc=False),
      scratch_types=[
          pltpu.VMEM((NCHK + 1, CH), jnp.int32),
          pltpu.VMEM((NCHK + 1, CH), jnp.int32),
          pltpu.VMEM((NBUF, CH, feat), jnp.float32),
          pltpu.VMEM((RP, feat), jnp.float32),
          pltpu.VMEM_SHARED((NP, feat), jnp.float32),
          pltpu.VMEM_SHARED((NP, feat), jnp.float32),
      ] + [pltpu.SemaphoreType.DMA] * (2 * NBUF),
  )
  def agg_kernel(h_hbm, ei_hbm, zeros_hbm, out_hbm, src_v, dst_v,
                 bufs, zb, acc_s, tab_s, *sems):
    gsem = sems[:NBUF]
    ssem = sems[NBUF:]
    c = lax.axis_index("c")
    s = lax.axis_index("s")
    wid = c * NS + s
    rows = pl.ds(s * RP, RP)

    pltpu.sync_copy(h_hbm.at[rows, pl.ds(0, feat)], zb)
    pltpu.sync_copy(zb, tab_s.at[rows])
    pltpu.sync_copy(zeros_hbm.at[:, pl.ds(0, feat)], zb)
    pltpu.sync_copy(zb, acc_s.at[rows])
    plsc.subcore_barrier()

    pltpu.sync_copy(ei_hbm.at[0, pl.ds(wid * NCHK, NCHK)],
                    src_v.at[pl.ds(0, NCHK)])
    pltpu.sync_copy(ei_hbm.at[1, pl.ds(wid * NCHK, NCHK)],
                    dst_v.at[pl.ds(0, NCHK)])

    def gissue(j, b):
      pltpu.async_copy(tab_s.at[src_v.at[j]], bufs.at[b], gsem[b])

    def gwait(b):
      pltpu.make_async_copy(tab_s.at[src_v.at[0]], bufs.at[b],
                            gsem[b]).wait()

    def sissue(j, b):
      pltpu.async_copy(bufs.at[b], acc_s.at[dst_v.at[j]], ssem[b], add=True)

    def swait(b):
      pltpu.make_async_copy(bufs.at[0], acc_s.at[dst_v.at[0]],
                            ssem[b]).wait()

    for j in range(PD):
      gissue(j, j)

    def visit(m, i, first_group, last_group):
      b = i
      gwait(b)
      sissue(m, b)
      bn = (i + PD) % NBUF
      if not (first_group and i < PD):
        swait(bn)
      if not (last_group and i >= NBUF - PD):
        gissue(m + PD, bn)

    for i in range(NBUF):
      visit(i, i, True, False)

    def body(g, carry):
      m0 = g * NBUF
      for i in range(NBUF):
        visit(m0 + i, i, False, False)
      return carry

    lax.fori_loop(1, NGRP - 1, body, 0)
    for i in range(NBUF):
      visit((NGRP - 1) * NBUF + i, i, False, True)
    for b in range(PD, NBUF):
      swait(b)

    @pl.when(wid < NEXTRA)
    def _():
      pltpu.sync_copy(ei_hbm.at[0, pl.ds(NCHK * NW + wid, 1)],
                      src_v.at[pl.ds(NCHK, 1)])
      pltpu.sync_copy(ei_hbm.at[1, pl.ds(NCHK * NW + wid, 1)],
                      dst_v.at[pl.ds(NCHK, 1)])
      gissue(NCHK, 0)
      gwait(0)
      sissue(NCHK, 0)
      swait(0)

    plsc.subcore_barrier()
    pltpu.sync_copy(acc_s.at[rows], zb)
    pltpu.sync_copy(zb, out_hbm.at[rows, pl.ds(c * feat, feat)])

  return agg_kernel


BR = 2560
GRID = NP // BR
def _tc1_body(x_ref, w_ref, dp_ref, h_ref, ddi_ref):
  deg = 1.0 + dp_ref[:, 0:1] + dp_ref[:, FD:FD + 1]
  dis = lax.rsqrt(deg)
  inv = 1.0 / deg
  ht = jnp.dot(x_ref[...], w_ref[...], preferred_element_type=jnp.float32)
  h_ref[...] = ht * dis
  col = lax.broadcasted_iota(jnp.int32, (BR, FD), 1)
  ddi_ref[...] = jnp.where(col == 0, deg, jnp.where(col == 1, dis, inv))


def _make_tc_mid_body(fin):
  def body(ap_ref, h_ref, ddi_ref, w_ref, b_ref, out_ref):
    dis = ddi_ref[:, 1:2]
    inv = ddi_ref[:, 2:3]
    acc = ap_ref[:, 0:fin] + ap_ref[:, fin:2 * fin] + h_ref[:, 0:fin]
    o = jnp.maximum(acc * dis * inv + b_ref[...], 0.0)
    out_ref[...] = jnp.dot(o, w_ref[...],
                           preferred_element_type=jnp.float32) * dis
  return body


def _tc4_body(ap_ref, h_ref, ddi_ref, b3_ref, wc_ref, bc_ref, out_ref):
  dis = ddi_ref[:, 1:2]
  acc = ap_ref[:, 0:F2] + ap_ref[:, F2:2 * F2] + h_ref[:, 0:F2]
  o = jnp.maximum(acc * dis + b3_ref[...], 0.0)
  logits = jnp.dot(o, wc_ref[...],
                   preferred_element_type=jnp.float32) + bc_ref[...]
  m = jnp.max(logits, axis=1, keepdims=True)
  e = jnp.exp(logits - m)
  lse = m + jnp.log(jnp.sum(e, axis=1, keepdims=True))
  out_ref[...] = logits - lse


def _row_spec(f):
  return pl.BlockSpec((BR, f), lambda i: (i, 0))


def _full_spec(r, f):
  return pl.BlockSpec((r, f), lambda i: (0, 0))


def _pad2(a, rows, cols):
  return jnp.pad(a, ((0, rows - a.shape[0]), (0, cols - a.shape[1])))


def kernel(x, edge_index, W1, b1, W2, b2, W3, b3, Wc, bc):
  f32 = jnp.float32
  nclass = Wc.shape[0]

  w1t = _pad2(W1.T, 128, 128)
  w2t = _pad2(W2.T, F1, 128)
  w3t = _pad2(W3.T, F2, 128)
  wct = _pad2(Wc.T, F2, nclass)
  b1p = _pad2(b1[None, :], 1, F1)
  b2p = _pad2(b2[None, :], 1, F2)
  b3p = _pad2(b3[None, :], 1, F2)
  bcp = bc[None, :]

  ei = edge_index.reshape(2, NROW, CH)
  zeros128 = jnp.zeros((RP, 128), f32)
  ones128 = jnp.ones((CH, 128), f32)

  degp = _make_deg_kernel()(ei, zeros128, ones128)

  h1p, ddi = pl.pallas_call(
      _tc1_body,
      grid=(GRID,),
      in_specs=[_row_spec(128), _full_spec(128, 128), _row_spec(128)],
      out_specs=[_row_spec(128), _row_spec(FD)],
      out_shape=[jax.ShapeDtypeStruct((NP, 128), f32),
                 jax.ShapeDtypeStruct((NP, FD), f32)],
  )(x, w1t, degp)

  acc1 = _make_agg_kernel(F1)(h1p, ei, zeros128)

  h2p = pl.pallas_call(
      _make_tc_mid_body(F1),
      grid=(GRID,),
      in_specs=[_row_spec(128), _row_spec(128), _row_spec(FD),
                _full_spec(F1, 128), _full_spec(1, F1)],
      out_specs=_row_spec(128),
      out_shape=jax.ShapeDtypeStruct((NP, 128), f32),
  )(acc1, h1p, ddi, w2t, b1p)

  acc2 = _make_agg_kernel(F2)(h2p, ei, zeros128)

  h3p = pl.pallas_call(
      _make_tc_mid_body(F2),
      grid=(GRID,),
      in_specs=[_row_spec(128), _row_spec(128), _row_spec(FD),
                _full_spec(F2, 128), _full_spec(1, F2)],
      out_specs=_row_spec(128),
      out_shape=jax.ShapeDtypeStruct((NP, 128), f32),
  )(acc2, h2p, ddi, w3t, b2p)

  acc3 = _make_agg_kernel(F2)(h3p, ei, zeros128)

  out = pl.pallas_call(
      _tc4_body,
      grid=(GRID,),
      in_specs=[_row_spec(128), _row_spec(128), _row_spec(FD),
                _full_spec(1, F2), _full_spec(F2, nclass),
                _full_spec(1, nclass)],
      out_specs=_row_spec(nclass),
      out_shape=jax.ShapeDtypeStruct((NNODE, nclass), f32),
  )(acc3, h3p, ddi, b3p, wct, bcp)

  return out


# speedup vs baseline: 80.3140x; 1.0046x over previous
import functools

import jax
import jax.numpy as jnp
from jax import lax
from jax.experimental import pallas as pl
from jax.experimental.pallas import tpu as pltpu
from jax.experimental.pallas import tpu_sc as plsc

NNODE = 10000
NEDGE = 320000
NP = 10240
F1 = 32
F2 = 8
FD = 8
NC, NS, L = 2, 16, 16
NW = NC * NS
CH = 128
NROW = NEDGE // CH
NCHK = NROW // NW
NEXTRA = NROW - NCHK * NW
RP = NP // NS

NBUF = 6
PD = 3
NGRP = NCHK // NBUF


@functools.lru_cache(maxsize=None)
def _mesh():
  return plsc.VectorSubcoreMesh(
      core_axis_name="c", subcore_axis_name="s", num_cores=NC, num_subcores=NS)


@functools.lru_cache(maxsize=None)
def _make_deg_kernel():
  @functools.partial(
      pl.kernel,
      out_type=jax.ShapeDtypeStruct((NP, 128), jnp.float32),
      mesh=_mesh(),
      compiler_params=pltpu.CompilerParams(use_tc_tiling_on_sc=False),
      scratch_types=[
          pltpu.VMEM((NCHK + 1, CH), jnp.int32),
          pltpu.VMEM((CH, FD), jnp.float32),
          pltpu.VMEM((RP, FD), jnp.float32),
          pltpu.VMEM_SHARED((NP, FD), jnp.float32),
      ] + [pltpu.SemaphoreType.DMA] * NBUF,
  )
  def deg_kernel(ei_hbm, zeros_hbm, ones_hbm, out_hbm, dst_v, ones_v, zb,
                 acc_s, *ssem):
    c = lax.axis_index("c")
    s = lax.axis_index("s")
    wid = c * NS + s
    rows = pl.ds(s * RP, RP)

    pltpu.sync_copy(ones_hbm.at[:, pl.ds(0, FD)], ones_v)
    pltpu.sync_copy(zeros_hbm.at[:, pl.ds(0, FD)], zb)
    pltpu.sync_copy(zb, acc_s.at[rows])
    plsc.subcore_barrier()

    pltpu.sync_copy(ei_hbm.at[1, pl.ds(wid * NCHK, NCHK)],
                    dst_v.at[pl.ds(0, NCHK)])

    def scat(j, b):
      pltpu.async_copy(ones_v, acc_s.at[dst_v.at[j]], ssem[b], add=True)

    def swait(b):
      pltpu.make_async_copy(ones_v, acc_s.at[dst_v.at[0]], ssem[b]).wait()

    for i in range(NBUF):
      scat(i, i)

    def body(g, carry):
      m0 = g * NBUF
      for i in range(NBUF):
        swait(i)
        scat(m0 + i, i)
      return carry

    lax.fori_loop(1, NGRP, body, 0)
    for i in range(NBUF):
      swait(i)

    @pl.when(wid < NEXTRA)
    def _():
      pltpu.sync_copy(ei_hbm.at[1, pl.ds(NCHK * NW + wid, 1)],
                      dst_v.at[pl.ds(NCHK, 1)])
      scat(NCHK, 0)
      swait(0)

    plsc.subcore_barrier()
    pltpu.sync_copy(acc_s.at[rows], zb)
    pltpu.sync_copy(zb, out_hbm.at[rows, pl.ds(c * FD, FD)])

  return deg_kernel


@functools.lru_cache(maxsize=None)
def _make_agg_kernel(feat):
  @functools.partial(
      pl.kernel,
      out_type=jax.ShapeDtypeStruct((NP, 128), jnp.float32),
      mesh=_mesh(),
      compiler_params=pltpu.CompilerParams(use_tc_tiling_on_sc=False),
      scratch_types=[
          pltpu.VMEM((NCHK + 1, CH), jnp.int32),
          pltpu.VMEM((NCHK + 1, CH), jnp.int32),
          pltpu.VMEM((NBUF, CH, feat), jnp.float32),
          pltpu.VMEM((RP, feat), jnp.float32),
          pltpu.VMEM_SHARED((NP, feat), jnp.float32),
          pltpu.VMEM_SHARED((NP, feat), jnp.float32),
      ] + [pltpu.SemaphoreType.DMA] * (2 * NBUF),
  )
  def agg_kernel(h_hbm, ei_hbm, zeros_hbm, out_hbm, src_v, dst_v,
                 bufs, zb, acc_s, tab_s, *sems):
    gsem = sems[:NBUF]
    ssem = sems[NBUF:]
    c = lax.axis_index("c")
    s = lax.axis_index("s")
    wid = c * NS + s
    rows = pl.ds(s * RP, RP)

    pltpu.sync_copy(h_hbm.at[rows, pl.ds(0, feat)], zb)
    pltpu.sync_copy(zb, tab_s.at[rows])
    pltpu.sync_copy(zeros_hbm.at[:, pl.ds(0, feat)], zb)
    pltpu.sync_copy(zb, acc_s.at[rows])
    plsc.subcore_barrier()

    pltpu.sync_copy(ei_hbm.at[0, pl.ds(wid * NCHK, NCHK)],
                    src_v.at[pl.ds(0, NCHK)])
    pltpu.sync_copy(ei_hbm.at[1, pl.ds(wid * NCHK, NCHK)],
                    dst_v.at[pl.ds(0, NCHK)])

    def gissue(j, b):
      pltpu.async_copy(tab_s.at[src_v.at[j]], bufs.at[b], gsem[b])

    def gwait(b):
      pltpu.make_async_copy(tab_s.at[src_v.at[0]], bufs.at[b],
                            gsem[b]).wait()

    def sissue(j, b):
      pltpu.async_copy(bufs.at[b], acc_s.at[dst_v.at[j]], ssem[b], add=True)

    def swait(b):
      pltpu.make_async_copy(bufs.at[0], acc_s.at[dst_v.at[0]],
                            ssem[b]).wait()

    for j in range(PD):
      gissue(j, j)

    def visit(m, i, first_group, last_group):
      b = i
      gwait(b)
      sissue(m, b)
      bn = (i + PD) % NBUF
      if not (first_group and i < PD):
        swait(bn)
      if not (last_group and i >= NBUF - PD):
        gissue(m + PD, bn)

    for i in range(NBUF):
      visit(i, i, True, False)

    def body(g, carry):
      m0 = g * NBUF
      for i in range(NBUF):
        visit(m0 + i, i, False, False)
      return carry

    lax.fori_loop(1, NGRP - 1, body, 0)
    for i in range(NBUF):
      visit((NGRP - 1) * NBUF + i, i, False, True)
    for b in range(PD, NBUF):
      swait(b)

    @pl.when(wid < NEXTRA)
    def _():
      pltpu.sync_copy(ei_hbm.at[0, pl.ds(NCHK * NW + wid, 1)],
                      src_v.at[pl.ds(NCHK, 1)])
      pltpu.sync_copy(ei_hbm.at[1, pl.ds(NCHK * NW + wid, 1)],
                      dst_v.at[pl.ds(NCHK, 1)])
      gissue(NCHK, 0)
      gwait(0)
      sissue(NCHK, 0)
      swait(0)

    plsc.subcore_barrier()
    pltpu.sync_copy(acc_s.at[rows], zb)
    pltpu.sync_copy(zb, out_hbm.at[rows, pl.ds(c * feat, feat)])

  return agg_kernel


BR = 5120
GRID = NP // BR
def _tc1_body(x_ref, w_ref, dp_ref, h_ref, ddi_ref):
  deg = 1.0 + dp_ref[:, 0:1] + dp_ref[:, FD:FD + 1]
  dis = lax.rsqrt(deg)
  inv = 1.0 / deg
  ht = jnp.dot(x_ref[...], w_ref[...], preferred_element_type=jnp.float32)
  h_ref[...] = ht * dis
  col = lax.broadcasted_iota(jnp.int32, (BR, FD), 1)
  ddi_ref[...] = jnp.where(col == 0, deg, jnp.where(col == 1, dis, inv))


def _make_tc_mid_body(fin):
  def body(ap_ref, h_ref, ddi_ref, w_ref, b_ref, out_ref):
    dis = ddi_ref[:, 1:2]
    inv = ddi_ref[:, 2:3]
    acc = ap_ref[:, 0:fin] + ap_ref[:, fin:2 * fin] + h_ref[:, 0:fin]
    o = jnp.maximum(acc * dis * inv + b_ref[...], 0.0)
    out_ref[...] = jnp.dot(o, w_ref[...],
                           preferred_element_type=jnp.float32) * dis
  return body


def _tc4_body(ap_ref, h_ref, ddi_ref, b3_ref, wc_ref, bc_ref, out_ref):
  dis = ddi_ref[:, 1:2]
  acc = ap_ref[:, 0:F2] + ap_ref[:, F2:2 * F2] + h_ref[:, 0:F2]
  o = jnp.maximum(acc * dis + b3_ref[...], 0.0)
  logits = jnp.dot(o, wc_ref[...],
                   preferred_element_type=jnp.float32) + bc_ref[...]
  m = jnp.max(logits, axis=1, keepdims=True)
  e = jnp.exp(logits - m)
  lse = m + jnp.log(jnp.sum(e, axis=1, keepdims=True))
  out_ref[...] = logits - lse


def _row_spec(f):
  return pl.BlockSpec((BR, f), lambda i: (i, 0))


def _full_spec(r, f):
  return pl.BlockSpec((r, f), lambda i: (0, 0))


def _pad2(a, rows, cols):
  return jnp.pad(a, ((0, rows - a.shape[0]), (0, cols - a.shape[1])))


def kernel(x, edge_index, W1, b1, W2, b2, W3, b3, Wc, bc):
  f32 = jnp.float32
  nclass = Wc.shape[0]

  w1t = _pad2(W1.T, 128, 128)
  w2t = _pad2(W2.T, F1, 128)
  w3t = _pad2(W3.T, F2, 128)
  wct = _pad2(Wc.T, F2, nclass)
  b1p = _pad2(b1[None, :], 1, F1)
  b2p = _pad2(b2[None, :], 1, F2)
  b3p = _pad2(b3[None, :], 1, F2)
  bcp = bc[None, :]

  ei = edge_index.reshape(2, NROW, CH)
  zeros128 = jnp.zeros((RP, 128), f32)
  ones128 = jnp.ones((CH, 128), f32)

  degp = _make_deg_kernel()(ei, zeros128, ones128)

  h1p, ddi = pl.pallas_call(
      _tc1_body,
      grid=(GRID,),
      in_specs=[_row_spec(128), _full_spec(128, 128), _row_spec(128)],
      out_specs=[_row_spec(128), _row_spec(FD)],
      out_shape=[jax.ShapeDtypeStruct((NP, 128), f32),
                 jax.ShapeDtypeStruct((NP, FD), f32)],
  )(x, w1t, degp)

  acc1 = _make_agg_kernel(F1)(h1p, ei, zeros128)

  h2p = pl.pallas_call(
      _make_tc_mid_body(F1),
      grid=(GRID,),
      in_specs=[_row_spec(128), _row_spec(128), _row_spec(FD),
                _full_spec(F1, 128), _full_spec(1, F1)],
      out_specs=_row_spec(128),
      out_shape=jax.ShapeDtypeStruct((NP, 128), f32),
  )(acc1, h1p, ddi, w2t, b1p)

  acc2 = _make_agg_kernel(F2)(h2p, ei, zeros128)

  h3p = pl.pallas_call(
      _make_tc_mid_body(F2),
      grid=(GRID,),
      in_specs=[_row_spec(128), _row_spec(128), _row_spec(FD),
                _full_spec(F2, 128), _full_spec(1, F2)],
      out_specs=_row_spec(128),
      out_shape=jax.ShapeDtypeStruct((NP, 128), f32),
  )(acc2, h2p, ddi, w3t, b2p)

  acc3 = _make_agg_kernel(F2)(h3p, ei, zeros128)

  out = pl.pallas_call(
      _tc4_body,
      grid=(GRID,),
      in_specs=[_row_spec(128), _row_spec(128), _row_spec(FD),
                _full_spec(1, F2), _full_spec(F2, nclass),
                _full_spec(1, nclass)],
      out_specs=_row_spec(nclass),
      out_shape=jax.ShapeDtypeStruct((NNODE, nclass), f32),
  )(acc3, h3p, ddi, b3p, wct, bcp)

  return out


# speedup vs baseline: 85.0986x; 1.0596x over previous
import functools

import jax
import jax.numpy as jnp
from jax import lax
from jax.experimental import pallas as pl
from jax.experimental.pallas import tpu as pltpu
from jax.experimental.pallas import tpu_sc as plsc

NNODE = 10000
NEDGE = 320000
NP = 10240
F1 = 32
F2 = 8
FD = 8
NC, NS, L = 2, 16, 16
NW = NC * NS
CH = 128
NROW = NEDGE // CH
NCHK = NROW // NW
NEXTRA = NROW - NCHK * NW
RP = NP // NS

NBUF = 6
PD = 3
NGRP = NCHK // NBUF


@functools.lru_cache(maxsize=None)
def _mesh():
  return plsc.VectorSubcoreMesh(
      core_axis_name="c", subcore_axis_name="s", num_cores=NC, num_subcores=NS)


@functools.lru_cache(maxsize=None)
def _make_deg_kernel():
  @functools.partial(
      pl.kernel,
      out_type=jax.ShapeDtypeStruct((NP, 128), jnp.float32),
      mesh=_mesh(),
      compiler_params=pltpu.CompilerParams(use_tc_tiling_on_sc=False),
      scratch_types=[
          pltpu.VMEM((NCHK + 1, CH), jnp.int32),
          pltpu.VMEM((CH, FD), jnp.float32),
          pltpu.VMEM((RP, FD), jnp.float32),
          pltpu.VMEM_SHARED((NP, FD), jnp.float32),
      ] + [pltpu.SemaphoreType.DMA] * NBUF,
  )
  def deg_kernel(ei_hbm, zeros_hbm, ones_hbm, out_hbm, dst_v, ones_v, zb,
                 acc_s, *ssem):
    c = lax.axis_index("c")
    s = lax.axis_index("s")
    wid = c * NS + s
    rows = pl.ds(s * RP, RP)

    pltpu.sync_copy(ones_hbm.at[:, pl.ds(0, FD)], ones_v)
    pltpu.sync_copy(zeros_hbm.at[:, pl.ds(0, FD)], zb)
    pltpu.sync_copy(zb, acc_s.at[rows])
    plsc.subcore_barrier()

    pltpu.sync_copy(ei_hbm.at[1, pl.ds(wid * NCHK, NCHK)],
                    dst_v.at[pl.ds(0, NCHK)])

    def scat(j, b):
      pltpu.async_copy(ones_v, acc_s.at[dst_v.at[j]], ssem[b], add=True)

    def swait(b):
      pltpu.make_async_copy(ones_v, acc_s.at[dst_v.at[0]], ssem[b]).wait()

    for i in range(NBUF):
      scat(i, i)

    def body(g, carry):
      m0 = g * NBUF
      for i in range(NBUF):
        swait(i)
        scat(m0 + i, i)
      return carry

    lax.fori_loop(1, NGRP, body, 0)
    for i in range(NBUF):
      swait(i)

    @pl.when(wid < NEXTRA)
    def _():
      pltpu.sync_copy(ei_hbm.at[1, pl.ds(NCHK * NW + wid, 1)],
                      dst_v.at[pl.ds(NCHK, 1)])
      scat(NCHK, 0)
      swait(0)

    plsc.subcore_barrier()
    pltpu.sync_copy(acc_s.at[rows], zb)
    pltpu.sync_copy(zb, out_hbm.at[rows, pl.ds(c * FD, FD)])

  return deg_kernel


@functools.lru_cache(maxsize=None)
def _make_agg_kernel(feat):
  @functools.partial(
      pl.kernel,
      out_type=jax.ShapeDtypeStruct((NP, 128), jnp.float32),
      mesh=_mesh(),
      compiler_params=pltpu.CompilerParams(use_tc_tiling_on_sc=False),
      scratch_types=[
          pltpu.VMEM((NCHK + 1, CH), jnp.int32),
          pltpu.VMEM((NCHK + 1, CH), jnp.int32),
          pltpu.VMEM((NBUF, CH, feat), jnp.float32),
          pltpu.VMEM((RP, feat), jnp.float32),
          pltpu.VMEM_SHARED((NP, feat), jnp.float32),
          pltpu.VMEM_SHARED((NP, feat), jnp.float32),
      ] + [pltpu.SemaphoreType.DMA] * (2 * NBUF),
  )
  def agg_kernel(h_hbm, ei_hbm, zeros_hbm, out_hbm, src_v, dst_v,
                 bufs, zb, acc_s, tab_s, *sems):
    gsem = sems[:NBUF]
    ssem = sems[NBUF:]
    c = lax.axis_index("c")
    s = lax.axis_index("s")
    wid = c * NS + s
    rows = pl.ds(s * RP, RP)

    pltpu.sync_copy(h_hbm.at[rows, pl.ds(0, feat)], zb)
    pltpu.sync_copy(zb, tab_s.at[rows])

    @pl.when(c == 0)
    def _():
      pltpu.sync_copy(zb, acc_s.at[rows])

    @pl.when(c == 1)
    def _():
      pltpu.sync_copy(zeros_hbm.at[:, pl.ds(0, feat)], zb)
      pltpu.sync_copy(zb, acc_s.at[rows])

    plsc.subcore_barrier()

    pltpu.sync_copy(ei_hbm.at[0, pl.ds(wid * NCHK, NCHK)],
                    src_v.at[pl.ds(0, NCHK)])
    pltpu.sync_copy(ei_hbm.at[1, pl.ds(wid * NCHK, NCHK)],
                    dst_v.at[pl.ds(0, NCHK)])

    def gissue(j, b):
      pltpu.async_copy(tab_s.at[src_v.at[j]], bufs.at[b], gsem[b])

    def gwait(b):
      pltpu.make_async_copy(tab_s.at[src_v.at[0]], bufs.at[b],
                            gsem[b]).wait()

    def sissue(j, b):
      pltpu.async_copy(bufs.at[b], acc_s.at[dst_v.at[j]], ssem[b], add=True)

    def swait(b):
      pltpu.make_async_copy(bufs.at[0], acc_s.at[dst_v.at[0]],
                            ssem[b]).wait()

    for j in range(PD):
      gissue(j, j)

    def visit(m, i, first_group, last_group):
      b = i
      gwait(b)
      sissue(m, b)
      bn = (i + PD) % NBUF
      if not (first_group and i < PD):
        swait(bn)
      if not (last_group and i >= NBUF - PD):
        gissue(m + PD, bn)

    for i in range(NBUF):
      visit(i, i, True, False)

    def body(g, carry):
      m0 = g * NBUF
      for i in range(NBUF):
        visit(m0 + i, i, False, False)
      return carry

    lax.fori_loop(1, NGRP - 1, body, 0)
    for i in range(NBUF):
      visit((NGRP - 1) * NBUF + i, i, False, True)
    for b in range(PD, NBUF):
      swait(b)

    @pl.when(wid < NEXTRA)
    def _():
      pltpu.sync_copy(ei_hbm.at[0, pl.ds(NCHK * NW + wid, 1)],
                      src_v.at[pl.ds(NCHK, 1)])
      pltpu.sync_copy(ei_hbm.at[1, pl.ds(NCHK * NW + wid, 1)],
                      dst_v.at[pl.ds(NCHK, 1)])
      gissue(NCHK, 0)
      gwait(0)
      sissue(NCHK, 0)
      swait(0)

    plsc.subcore_barrier()
    pltpu.sync_copy(acc_s.at[rows], zb)
    pltpu.sync_copy(zb, out_hbm.at[rows, pl.ds(c * feat, feat)])

  return agg_kernel


BR = 5120
GRID = NP // BR
def _tc1_body(x_ref, w_ref, dp_ref, h_ref, ddi_ref):
  deg = 1.0 + dp_ref[:, 0:1] + dp_ref[:, FD:FD + 1]
  dis = lax.rsqrt(deg)
  inv = 1.0 / deg
  ht = jnp.dot(x_ref[...], w_ref[...], preferred_element_type=jnp.float32)
  h_ref[...] = ht * dis
  col = lax.broadcasted_iota(jnp.int32, (BR, FD), 1)
  ddi_ref[...] = jnp.where(col == 0, deg, jnp.where(col == 1, dis, inv))


def _make_tc_mid_body(fin):
  def body(ap_ref, ddi_ref, w_ref, b_ref, out_ref):
    dis = ddi_ref[:, 1:2]
    inv = ddi_ref[:, 2:3]
    acc = ap_ref[:, 0:fin] + ap_ref[:, fin:2 * fin]
    o = jnp.maximum(acc * dis * inv + b_ref[...], 0.0)
    out_ref[...] = jnp.dot(o, w_ref[...],
                           preferred_element_type=jnp.float32) * dis
  return body


def _tc4_body(ap_ref, ddi_ref, b3_ref, wc_ref, bc_ref, out_ref):
  dis = ddi_ref[:, 1:2]
  acc = ap_ref[:, 0:F2] + ap_ref[:, F2:2 * F2]
  o = jnp.maximum(acc * dis + b3_ref[...], 0.0)
  logits = jnp.dot(o, wc_ref[...],
                   preferred_element_type=jnp.float32) + bc_ref[...]
  m = jnp.max(logits, axis=1, keepdims=True)
  e = jnp.exp(logits - m)
  lse = m + jnp.log(jnp.sum(e, axis=1, keepdims=True))
  out_ref[...] = logits - lse


def _row_spec(f):
  return pl.BlockSpec((BR, f), lambda i: (i, 0))


def _full_spec(r, f):
  return pl.BlockSpec((r, f), lambda i: (0, 0))


def _pad2(a, rows, cols):
  return jnp.pad(a, ((0, rows - a.shape[0]), (0, cols - a.shape[1])))


def kernel(x, edge_index, W1, b1, W2, b2, W3, b3, Wc, bc):
  f32 = jnp.float32
  nclass = Wc.shape[0]

  w1t = _pad2(W1.T, 128, 128)
  w2t = _pad2(W2.T, F1, 128)
  w3t = _pad2(W3.T, F2, 128)
  wct = _pad2(Wc.T, F2, nclass)
  b1p = _pad2(b1[None, :], 1, F1)
  b2p = _pad2(b2[None, :], 1, F2)
  b3p = _pad2(b3[None, :], 1, F2)
  bcp = bc[None, :]

  ei = edge_index.reshape(2, NROW, CH)
  zeros128 = jnp.zeros((RP, 128), f32)
  ones128 = jnp.ones((CH, 128), f32)

  degp = _make_deg_kernel()(ei, zeros128, ones128)

  h1p, ddi = pl.pallas_call(
      _tc1_body,
      grid=(GRID,),
      in_specs=[_row_spec(128), _full_spec(128, 128), _row_spec(128)],
      out_specs=[_row_spec(128), _row_spec(FD)],
      out_shape=[jax.ShapeDtypeStruct((NP, 128), f32),
                 jax.ShapeDtypeStruct((NP, FD), f32)],
  )(x, w1t, degp)

  acc1 = _make_agg_kernel(F1)(h1p, ei, zeros128)

  h2p = pl.pallas_call(
      _make_tc_mid_body(F1),
      grid=(GRID,),
      in_specs=[_row_spec(128), _row_spec(FD),
                _full_spec(F1, 128), _full_spec(1, F1)],
      out_specs=_row_spec(128),
      out_shape=jax.ShapeDtypeStruct((NP, 128), f32),
  )(acc1, ddi, w2t, b1p)

  acc2 = _make_agg_kernel(F2)(h2p, ei, zeros128)

  h3p = pl.pallas_call(
      _make_tc_mid_body(F2),
      grid=(GRID,),
      in_specs=[_row_spec(128), _row_spec(FD),
                _full_spec(F2, 128), _full_spec(1, F2)],
      out_specs=_row_spec(128),
      out_shape=jax.ShapeDtypeStruct((NP, 128), f32),
  )(acc2, ddi, w3t, b2p)

  acc3 = _make_agg_kernel(F2)(h3p, ei, zeros128)

  out = pl.pallas_call(
      _tc4_body,
      grid=(GRID,),
      in_specs=[_row_spec(128), _row_spec(FD),
                _full_spec(1, F2), _full_spec(F2, nclass),
                _full_spec(1, nclass)],
      out_specs=_row_spec(nclass),
      out_shape=jax.ShapeDtypeStruct((NNODE, nclass), f32),
  )(acc3, ddi, b3p, wct, bcp)

  return out
